# Initial kernel scaffold; baseline (speedup 1.0000x reference)
import jax
import jax.numpy as jnp
from jax.experimental import pallas as pl


def kernel(x, edge_index, batch, data_fp, W1, b1, W2, b2, W3, b3, Wg1, bg1, Wg2, bg2, Wf1, bf1, Wf2, bf2, Wo, bo):
    raise NotImplementedError("write your pallas kernel here")


# speedup vs baseline: 21.1567x; 21.1567x over previous
import functools

import jax
import jax.numpy as jnp
from jax import lax
from jax.experimental import pallas as pl
from jax.experimental.pallas import tpu as pltpu
from jax.experimental.pallas import tpu_sc as plsc

N = 10000
NP = 10240
E = 320000
EP = 327680
ER = EP // 128
RPT = ER // 16
RPT_H = ER // 32
NB = 32
RB = 1024
NBLK = NP // RB

_f32 = jnp.float32
_SDS = jax.ShapeDtypeStruct


def _sc_mesh():
  return plsc.VectorSubcoreMesh(core_axis_name="c", subcore_axis_name="s")


def _deg_body(dst2, zeros_np, out_a, out_b, sdeg, dstbuf, ones_v):
  c = lax.axis_index("c")
  s = lax.axis_index("s")
  for i in range(8):
    ones_v[pl.ds(i * 16, 16)] = jnp.full((16,), 1.0, _f32)
  pltpu.sync_copy(zeros_np.at[pl.ds(s * 640, 640)], sdeg.at[pl.ds(s * 640, 640)])
  pltpu.sync_copy(dst2.at[pl.ds(c * (RPT_H * 16) + s * RPT_H, RPT_H)], dstbuf)
  plsc.subcore_barrier()

  def step(j, carry):
    pltpu.sync_copy(ones_v, sdeg.at[dstbuf.at[j]], add=True)
    return carry

  lax.fori_loop(0, RPT_H, step, 0)
  plsc.subcore_barrier()

  @pl.when(c == 0)
  def _():
    pltpu.sync_copy(sdeg.at[pl.ds(s * 640, 640)], out_a.at[pl.ds(s * 640, 640)])

  @pl.when(c == 1)
  def _():
    pltpu.sync_copy(sdeg.at[pl.ds(s * 640, 640)], out_b.at[pl.ds(s * 640, 640)])


@functools.cache
def _deg_kernel():
  return pl.kernel(
      _deg_body,
      out_type=(_SDS((NP,), _f32), _SDS((NP,), _f32)),
      mesh=_sc_mesh(),
      scratch_types=[
          pltpu.VMEM_SHARED((NP,), _f32),
          pltpu.VMEM((RPT_H, 128), jnp.int32),
          pltpu.VMEM((128,), _f32),
      ],
  )


IDXB = 40


def _edge_pass(g_ref, acc, src2, dst2, base, nrows,
               srcblk, dstblk, rows0, rows1, sem0, sem1):
  def blk(b, carry):
    off = base + b * IDXB
    pltpu.sync_copy(src2.at[pl.ds(off, IDXB)], srcblk)
    pltpu.sync_copy(dst2.at[pl.ds(off, IDXB)], dstblk)

    def step(j, c2):
      cp0 = pltpu.async_copy(g_ref.at[srcblk.at[2 * j]], rows0, sem0)
      cp1 = pltpu.async_copy(g_ref.at[srcblk.at[2 * j + 1]], rows1, sem1)
      cp0.wait()
      pltpu.sync_copy(rows0, acc.at[dstblk.at[2 * j]], add=True)
      cp1.wait()
      pltpu.sync_copy(rows1, acc.at[dstblk.at[2 * j + 1]], add=True)
      return c2

    lax.fori_loop(0, IDXB // 2, step, 0)
    return carry

  lax.fori_loop(0, nrows // IDXB, blk, 0)


def _agg_split_body(g, src2, dst2, out0, out1,
                    acc, srcbuf, dstbuf, rows0, rows1, sem0, sem1):
  c = lax.axis_index("c")
  s = lax.axis_index("s")
  pltpu.sync_copy(g.at[pl.ds(s * 640, 640)], acc.at[pl.ds(s * 640, 640)])
  plsc.subcore_barrier()
  _edge_pass(g, acc, src2, dst2, c * (RPT_H * 16) + s * RPT_H, RPT_H,
             srcbuf, dstbuf, rows0, rows1, sem0, sem1)
  plsc.subcore_barrier()

  @pl.when(c == 0)
  def _():
    pltpu.sync_copy(acc.at[pl.ds(s * 640, 640)], out0.at[pl.ds(s * 640, 640)])

  @pl.when(c == 1)
  def _():
    pltpu.sync_copy(acc.at[pl.ds(s * 640, 640)], out1.at[pl.ds(s * 640, 640)])


def _agg_dual_body(g_a, g_b, src2, dst2, out_a, out_b,
                   acc, srcbuf, dstbuf, rows0, rows1, sem0, sem1):
  c = lax.axis_index("c")
  s = lax.axis_index("s")

  def run(g_ref, out_ref):
    pltpu.sync_copy(g_ref.at[pl.ds(s * 640, 640)], acc.at[pl.ds(s * 640, 640)])
    plsc.subcore_barrier()
    _edge_pass(g_ref, acc, src2, dst2, s * RPT, RPT,
               srcbuf, dstbuf, rows0, rows1, sem0, sem1)
    plsc.subcore_barrier()
    pltpu.sync_copy(acc.at[pl.ds(s * 640, 640)], out_ref.at[pl.ds(s * 640, 640)])

  @pl.when(c == 0)
  def _():
    run(g_a, out_a)

  @pl.when(c == 1)
  def _():
    run(g_b, out_b)


def _agg_scratch():
  return [
      pltpu.VMEM_SHARED((NP, 128), _f32),
      pltpu.VMEM((IDXB, 128), jnp.int32),
      pltpu.VMEM((IDXB, 128), jnp.int32),
      pltpu.VMEM((128, 128), _f32),
      pltpu.VMEM((128, 128), _f32),
      pltpu.SemaphoreType.DMA,
      pltpu.SemaphoreType.DMA,
  ]


@functools.cache
def _agg_split_kernel():
  return pl.kernel(
      _agg_split_body,
      out_type=(_SDS((NP, 128), _f32), _SDS((NP, 128), _f32)),
      mesh=_sc_mesh(),
      scratch_types=_agg_scratch(),
  )


@functools.cache
def _agg_dual_kernel():
  return pl.kernel(
      _agg_dual_body,
      out_type=(_SDS((NP, 128), _f32), _SDS((NP, 128), _f32)),
      mesh=_sc_mesh(),
      scratch_types=_agg_scratch(),
  )


def _row_mask(i):
  rid = i * RB + lax.broadcasted_iota(jnp.int32, (RB, 1), 0)
  return (rid < N).astype(_f32)


def _k1_body(dega_ref, degb_ref, x_ref, dinv_ref, g_ref):
  i = pl.program_id(0)
  deg = dega_ref[...] + degb_ref[...] + 1.0
  dinv = lax.rsqrt(jnp.maximum(deg, 1.0))
  dinv_ref[...] = dinv
  g_ref[...] = x_ref[...] * dinv * _row_mask(i)


def _k1(dega, degb, x_pad):
  return pl.pallas_call(
      _k1_body,
      grid=(NBLK,),
      in_specs=[
          pl.BlockSpec((RB, 1), lambda i: (i, 0)),
          pl.BlockSpec((RB, 1), lambda i: (i, 0)),
          pl.BlockSpec((RB, 128), lambda i: (i, 0)),
      ],
      out_specs=[
          pl.BlockSpec((RB, 1), lambda i: (i, 0)),
          pl.BlockSpec((RB, 128), lambda i: (i, 0)),
      ],
      out_shape=[_SDS((NP, 1), _f32), _SDS((NP, 128), _f32)],
  )(dega, degb, x_pad)


def _k2_body(p0_ref, p1_ref, g_ref, dinv_ref, w_ref, b_ref, out_ref):
  i = pl.program_id(0)
  d = dinv_ref[...]
  sm = (p0_ref[...] + p1_ref[...] - g_ref[...]) * d
  t = jnp.dot(sm, w_ref[...], preferred_element_type=_f32) + b_ref[...]
  out_ref[...] = jnp.maximum(t, 0.0) * d * _row_mask(i)


def _k2(p0, p1, g0, dinv, w1, b1):
  return pl.pallas_call(
      _k2_body,
      grid=(NBLK,),
      in_specs=[
          pl.BlockSpec((RB, 128), lambda i: (i, 0)),
          pl.BlockSpec((RB, 128), lambda i: (i, 0)),
          pl.BlockSpec((RB, 128), lambda i: (i, 0)),
          pl.BlockSpec((RB, 1), lambda i: (i, 0)),
          pl.BlockSpec((128, 128), lambda i: (0, 0)),
          pl.BlockSpec((1, 128), lambda i: (0, 0)),
      ],
      out_specs=pl.BlockSpec((RB, 128), lambda i: (i, 0)),
      out_shape=_SDS((NP, 128), _f32),
  )(p0, p1, g0, dinv, w1, b1)


def _k3_body(p0_ref, p1_ref, g_ref, dinv_ref, w2_ref, b2_ref, w3_ref,
             ga_ref, gb_ref, gc_ref):
  i = pl.program_id(0)
  d = dinv_ref[...]
  sm = (p0_ref[...] + p1_ref[...] - g_ref[...]) * d
  u = jnp.dot(sm, w2_ref[...], preferred_element_type=_f32) + b2_ref[...]
  u = jnp.maximum(u, 0.0)
  h = jnp.dot(u, w3_ref[...], preferred_element_type=_f32)
  g = h * d * _row_mask(i)
  ga_ref[...] = g[:, :128]
  gb_ref[...] = g[:, 128:256]
  gc_ref[...] = g[:, 192:]


def _k3(p0, p1, g1, dinv, w2, b2, w3):
  return pl.pallas_call(
      _k3_body,
      grid=(NBLK,),
      in_specs=[
          pl.BlockSpec((RB, 128), lambda i: (i, 0)),
          pl.BlockSpec((RB, 128), lambda i: (i, 0)),
          pl.BlockSpec((RB, 128), lambda i: (i, 0)),
          pl.BlockSpec((RB, 1), lambda i: (i, 0)),
          pl.BlockSpec((128, 640), lambda i: (0, 0)),
          pl.BlockSpec((1, 640), lambda i: (0, 0)),
          pl.BlockSpec((640, 320), lambda i: (0, 0)),
      ],
      out_specs=[
          pl.BlockSpec((RB, 128), lambda i: (i, 0)),
          pl.BlockSpec((RB, 128), lambda i: (i, 0)),
          pl.BlockSpec((RB, 128), lambda i: (i, 0)),
      ],
      out_shape=[_SDS((NP, 128), _f32)] * 3,
  )(p0, p1, g1, dinv, w2, b2, w3)


def _k4_body(oa_ref, ob_ref, q0_ref, q1_ref, gc_ref, dinv_ref, b3_ref,
             batch_ref, sums_ref, cnt_ref):
  i = pl.program_id(0)
  d = dinv_ref[...]
  sc2 = q0_ref[...] + q1_ref[...] - gc_ref[...]
  s2 = jnp.concatenate([oa_ref[...], ob_ref[:, :64], sc2], axis=1)
  x3 = jnp.maximum(s2 * d + b3_ref[...], 0.0)
  seg = lax.broadcasted_iota(jnp.int32, (RB, NB), 1)
  oh = (batch_ref[...] == seg).astype(_f32)
  psum = lax.dot_general(oh, x3, (((0,), (0,)), ((), ())),
                         preferred_element_type=_f32)
  pcnt = lax.dot_general(oh, jnp.ones((RB, 128), _f32), (((0,), (0,)), ((), ())),
                         preferred_element_type=_f32)

  @pl.when(i == 0)
  def _():
    sums_ref[...] = psum
    cnt_ref[...] = pcnt

  @pl.when(i > 0)
  def _():
    sums_ref[...] += psum
    cnt_ref[...] += pcnt


def _k4(oa, ob, q0, q1, g2c, dinv, b3, batch1):
  return pl.pallas_call(
      _k4_body,
      grid=(NBLK,),
      in_specs=[
          pl.BlockSpec((RB, 128), lambda i: (i, 0)),
          pl.BlockSpec((RB, 128), lambda i: (i, 0)),
          pl.BlockSpec((RB, 128), lambda i: (i, 0)),
          pl.BlockSpec((RB, 128), lambda i: (i, 0)),
          pl.BlockSpec((RB, 128), lambda i: (i, 0)),
          pl.BlockSpec((RB, 1), lambda i: (i, 0)),
          pl.BlockSpec((1, 320), lambda i: (0, 0)),
          pl.BlockSpec((RB, 1), lambda i: (i, 0)),
      ],
      out_specs=[
          pl.BlockSpec((NB, 320), lambda i: (0, 0)),
          pl.BlockSpec((NB, 128), lambda i: (0, 0)),
      ],
      out_shape=[_SDS((NB, 320), _f32), _SDS((NB, 128), _f32)],
      compiler_params=pltpu.CompilerParams(dimension_semantics=("arbitrary",)),
  )(oa, ob, q0, q1, g2c, dinv, b3, batch1)


def _k5_body(sums_ref, cnt_ref, fp_ref, wg1_ref, bg1_ref, wg2_ref, bg2_ref,
             wf1a_ref, wf1b_ref, bf1_ref, wf2_ref, bf2_ref, wo_ref, bo_ref,
             out_ref):
  cnt = jnp.maximum(cnt_ref[:, :1], 1.0)
  mean = sums_ref[...] / cnt
  z1 = jnp.maximum(jnp.dot(mean, wg1_ref[...], preferred_element_type=_f32)
                   + bg1_ref[...], 0.0)
  z2 = jnp.dot(z1, wg2_ref[...], preferred_element_type=_f32) + bg2_ref[...]
  t = (jnp.dot(fp_ref[...], wf1a_ref[...], preferred_element_type=_f32)
       + jnp.dot(z2, wf1b_ref[...], preferred_element_type=_f32)
       + bf1_ref[...])
  t = jnp.maximum(t, 0.0)
  t2 = jnp.maximum(jnp.dot(t, wf2_ref[...], preferred_element_type=_f32)
                   + bf2_ref[...], 0.0)
  out_ref[...] = (jnp.dot(t2, wo_ref[...], preferred_element_type=_f32)
                  + bo_ref[...])


def _k5(sums, cnt, fp_pad, wg1, bg1, wg2, bg2, wf1a, wf1b, bf1, wf2, bf2, wo, bo):
  return pl.pallas_call(
      _k5_body,
      out_shape=_SDS((NB, 1), _f32),
  )(sums, cnt, fp_pad, wg1, bg1, wg2, bg2, wf1a, wf1b, bf1, wf2, bf2, wo, bo)


def kernel(x, edge_index, batch, data_fp, W1, b1, W2, b2, W3, b3,
           Wg1, bg1, Wg2, bg2, Wf1, bf1, Wf2, bf2, Wo, bo):
  pad_idx = (N + (jnp.arange(EP - E, dtype=jnp.int32) % (NP - N))).astype(jnp.int32)
  src2 = jnp.concatenate([edge_index[0], pad_idx]).reshape(ER, 128)
  dst2 = jnp.concatenate([edge_index[1], pad_idx]).reshape(ER, 128)
  x_pad = jnp.pad(x, ((0, NP - N), (0, 0)))
  batch1 = jnp.pad(batch, (0, NP - N), constant_values=NB).reshape(NP, 1)
  zeros_np = jnp.zeros((NP,), _f32)
  fp_pad = jnp.pad(data_fp.astype(_f32), ((0, 0), (0, 63)))
  wf1a = jnp.pad(Wf1[:65], ((0, 63), (0, 0)))
  wf1b = Wf1[65:]
  b1r = b1.reshape(1, 128)
  b2r = b2.reshape(1, 640)
  b3r = b3.reshape(1, 320)
  bg1r, bg2r = bg1.reshape(1, 1024), bg2.reshape(1, 128)
  bf1r, bf2r = bf1.reshape(1, 1024), bf2.reshape(1, 512)
  bor = bo.reshape(1, 1)

  dega, degb = _deg_kernel()(dst2, zeros_np)
  dinv, g0 = _k1(dega.reshape(NP, 1), degb.reshape(NP, 1), x_pad)

  p0, p1 = _agg_split_kernel()(g0, src2, dst2)
  g1 = _k2(p0, p1, g0, dinv, W1, b1r)

  p0, p1 = _agg_split_kernel()(g1, src2, dst2)
  g2a, g2b, g2c = _k3(p0, p1, g1, dinv, W2, b2r, W3)

  oa, ob = _agg_dual_kernel()(g2a, g2b, src2, dst2)
  q0, q1 = _agg_split_kernel()(g2c, src2, dst2)
  sums, cnt = _k4(oa, ob, q0, q1, g2c, dinv, b3r, batch1)

  return _k5(sums, cnt, fp_pad, Wg1, bg1r, Wg2, bg2r,
             wf1a, wf1b, bf1r, Wf2, bf2r, Wo, bor)


# speedup vs baseline: 27.0687x; 1.2794x over previous
import functools

import jax
import jax.numpy as jnp
from jax import lax
from jax.experimental import pallas as pl
from jax.experimental.pallas import tpu as pltpu
from jax.experimental.pallas import tpu_sc as plsc

N = 10000
NP = 10240
E = 320000
EP = 327680
ER = EP // 64
RPT = ER // 16
RPT_H = ER // 32
NB = 32
RB = 1024
NBLK = NP // RB

_f32 = jnp.float32
_SDS = jax.ShapeDtypeStruct


def _sc_mesh():
  return plsc.VectorSubcoreMesh(core_axis_name="c", subcore_axis_name="s")


def _deg_body(dst3, zeros_np, out_a, out_b, sdeg, dstbuf, ones_v):
  c = lax.axis_index("c")
  s = lax.axis_index("s")
  for i in range(4):
    ones_v[pl.ds(i * 16, 16)] = jnp.full((16,), 1.0, _f32)
  pltpu.sync_copy(zeros_np.at[pl.ds(s * 640, 640)], sdeg.at[pl.ds(s * 640, 640)])
  pltpu.sync_copy(dst3.at[pl.ds(c * (RPT_H * 16) + s * RPT_H, RPT_H)], dstbuf)
  plsc.subcore_barrier()

  def step(j, carry):
    pltpu.sync_copy(ones_v, sdeg.at[dstbuf.at[j]], add=True)
    return carry

  lax.fori_loop(0, RPT_H, step, 0)
  plsc.subcore_barrier()

  @pl.when(c == 0)
  def _():
    pltpu.sync_copy(sdeg.at[pl.ds(s * 640, 640)], out_a.at[pl.ds(s * 640, 640)])

  @pl.when(c == 1)
  def _():
    pltpu.sync_copy(sdeg.at[pl.ds(s * 640, 640)], out_b.at[pl.ds(s * 640, 640)])


@functools.cache
def _deg_kernel():
  return pl.kernel(
      _deg_body,
      out_type=(_SDS((NP,), _f32), _SDS((NP,), _f32)),
      mesh=_sc_mesh(),
      scratch_types=[
          pltpu.VMEM_SHARED((NP,), _f32),
          pltpu.VMEM((RPT_H, W), jnp.int32),
          pltpu.VMEM((W,), _f32),
      ],
  )


W = 64
IDXB = 40
NBUF = 4


def _edge_pass(g_ref, acc, src3, dst3, base, nrows,
               srcblk, dstblk, bufs, gsems, ssems):
  def blk(bi, carry):
    off = base + bi * IDXB
    pltpu.sync_copy(src3.at[pl.ds(off, IDXB)], srcblk)
    pltpu.sync_copy(dst3.at[pl.ds(off, IDXB)], dstblk)
    pltpu.async_copy(g_ref.at[srcblk.at[0]], bufs[0], gsems[0])
    pltpu.async_copy(g_ref.at[srcblk.at[1]], bufs[1], gsems[1])

    def grp(gi, c2):
      for b in range(NBUF):
        j = NBUF * gi + b
        b2 = (b + 2) % NBUF

        @pl.when(j + 2 < IDXB)
        def _():
          @pl.when(j >= 2)
          def _():
            pltpu.make_async_copy(bufs[b2], acc.at[dstblk.at[j]],
                                  ssems[b2]).wait()
          pltpu.async_copy(g_ref.at[srcblk.at[j + 2]], bufs[b2], gsems[b2])

        pltpu.make_async_copy(g_ref.at[srcblk.at[j]], bufs[b], gsems[b]).wait()
        pltpu.async_copy(bufs[b], acc.at[dstblk.at[j]], ssems[b], add=True)
      return c2

    lax.fori_loop(0, IDXB // NBUF, grp, 0)
    for b in range(NBUF):
      pltpu.make_async_copy(bufs[b], acc.at[dstblk.at[0]], ssems[b]).wait()
    return carry

  lax.fori_loop(0, nrows // IDXB, blk, 0)


def _agg_split_body(g, src3, dst3, out0, out1,
                    acc, srcblk, dstblk,
                    b0, b1, b2, b3, ga0, ga1, ga2, ga3, sa0, sa1, sa2, sa3):
  c = lax.axis_index("c")
  s = lax.axis_index("s")
  pltpu.sync_copy(g.at[pl.ds(s * 640, 640)], acc.at[pl.ds(s * 640, 640)])
  plsc.subcore_barrier()
  _edge_pass(g, acc, src3, dst3, c * (RPT_H * 16) + s * RPT_H, RPT_H,
             srcblk, dstblk, [b0, b1, b2, b3],
             [ga0, ga1, ga2, ga3], [sa0, sa1, sa2, sa3])
  plsc.subcore_barrier()

  @pl.when(c == 0)
  def _():
    pltpu.sync_copy(acc.at[pl.ds(s * 640, 640)], out0.at[pl.ds(s * 640, 640)])

  @pl.when(c == 1)
  def _():
    pltpu.sync_copy(acc.at[pl.ds(s * 640, 640)], out1.at[pl.ds(s * 640, 640)])


def _agg_dual_body(g_a, g_b, src3, dst3, out_a, out_b,
                   acc, srcblk, dstblk,
                   b0, b1, b2, b3, ga0, ga1, ga2, ga3, sa0, sa1, sa2, sa3):
  c = lax.axis_index("c")
  s = lax.axis_index("s")

  def run(g_ref, out_ref):
    pltpu.sync_copy(g_ref.at[pl.ds(s * 640, 640)], acc.at[pl.ds(s * 640, 640)])
    plsc.subcore_barrier()
    _edge_pass(g_ref, acc, src3, dst3, s * RPT, RPT,
               srcblk, dstblk, [b0, b1, b2, b3],
               [ga0, ga1, ga2, ga3], [sa0, sa1, sa2, sa3])
    plsc.subcore_barrier()
    pltpu.sync_copy(acc.at[pl.ds(s * 640, 640)], out_ref.at[pl.ds(s * 640, 640)])

  @pl.when(c == 0)
  def _():
    run(g_a, out_a)

  @pl.when(c == 1)
  def _():
    run(g_b, out_b)


def _agg_scratch():
  return ([pltpu.VMEM_SHARED((NP, 128), _f32),
           pltpu.VMEM((IDXB, W), jnp.int32),
           pltpu.VMEM((IDXB, W), jnp.int32)]
          + [pltpu.VMEM((W, 128), _f32)] * NBUF
          + [pltpu.SemaphoreType.DMA] * (2 * NBUF))


@functools.cache
def _agg_split_kernel():
  return pl.kernel(
      _agg_split_body,
      out_type=(_SDS((NP, 128), _f32), _SDS((NP, 128), _f32)),
      mesh=_sc_mesh(),
      scratch_types=_agg_scratch(),
  )


@functools.cache
def _agg_dual_kernel():
  return pl.kernel(
      _agg_dual_body,
      out_type=(_SDS((NP, 128), _f32), _SDS((NP, 128), _f32)),
      mesh=_sc_mesh(),
      scratch_types=_agg_scratch(),
  )


def _row_mask(i):
  rid = i * RB + lax.broadcasted_iota(jnp.int32, (RB, 1), 0)
  return (rid < N).astype(_f32)


def _k1_body(dega_ref, degb_ref, x_ref, dinv_ref, g_ref):
  i = pl.program_id(0)
  deg = dega_ref[...] + degb_ref[...] + 1.0
  dinv = lax.rsqrt(jnp.maximum(deg, 1.0))
  dinv_ref[...] = dinv
  g_ref[...] = x_ref[...] * dinv * _row_mask(i)


def _k1(dega, degb, x_pad):
  return pl.pallas_call(
      _k1_body,
      grid=(NBLK,),
      in_specs=[
          pl.BlockSpec((RB, 1), lambda i: (i, 0)),
          pl.BlockSpec((RB, 1), lambda i: (i, 0)),
          pl.BlockSpec((RB, 128), lambda i: (i, 0)),
      ],
      out_specs=[
          pl.BlockSpec((RB, 1), lambda i: (i, 0)),
          pl.BlockSpec((RB, 128), lambda i: (i, 0)),
      ],
      out_shape=[_SDS((NP, 1), _f32), _SDS((NP, 128), _f32)],
  )(dega, degb, x_pad)


def _k2_body(p0_ref, p1_ref, g_ref, dinv_ref, w_ref, b_ref, out_ref):
  i = pl.program_id(0)
  d = dinv_ref[...]
  sm = (p0_ref[...] + p1_ref[...] - g_ref[...]) * d
  t = jnp.dot(sm, w_ref[...], preferred_element_type=_f32) + b_ref[...]
  out_ref[...] = jnp.maximum(t, 0.0) * d * _row_mask(i)


def _k2(p0, p1, g0, dinv, w1, b1):
  return pl.pallas_call(
      _k2_body,
      grid=(NBLK,),
      in_specs=[
          pl.BlockSpec((RB, 128), lambda i: (i, 0)),
          pl.BlockSpec((RB, 128), lambda i: (i, 0)),
          pl.BlockSpec((RB, 128), lambda i: (i, 0)),
          pl.BlockSpec((RB, 1), lambda i: (i, 0)),
          pl.BlockSpec((128, 128), lambda i: (0, 0)),
          pl.BlockSpec((1, 128), lambda i: (0, 0)),
      ],
      out_specs=pl.BlockSpec((RB, 128), lambda i: (i, 0)),
      out_shape=_SDS((NP, 128), _f32),
  )(p0, p1, g0, dinv, w1, b1)


def _k3_body(p0_ref, p1_ref, g_ref, dinv_ref, w2_ref, b2_ref, w3_ref,
             ga_ref, gb_ref, gc_ref):
  i = pl.program_id(0)
  d = dinv_ref[...]
  sm = (p0_ref[...] + p1_ref[...] - g_ref[...]) * d
  u = jnp.dot(sm, w2_ref[...], preferred_element_type=_f32) + b2_ref[...]
  u = jnp.maximum(u, 0.0)
  h = jnp.dot(u, w3_ref[...], preferred_element_type=_f32)
  g = h * d * _row_mask(i)
  ga_ref[...] = g[:, :128]
  gb_ref[...] = g[:, 128:256]
  gc_ref[...] = g[:, 192:]


def _k3(p0, p1, g1, dinv, w2, b2, w3):
  return pl.pallas_call(
      _k3_body,
      grid=(NBLK,),
      in_specs=[
          pl.BlockSpec((RB, 128), lambda i: (i, 0)),
          pl.BlockSpec((RB, 128), lambda i: (i, 0)),
          pl.BlockSpec((RB, 128), lambda i: (i, 0)),
          pl.BlockSpec((RB, 1), lambda i: (i, 0)),
          pl.BlockSpec((128, 640), lambda i: (0, 0)),
          pl.BlockSpec((1, 640), lambda i: (0, 0)),
          pl.BlockSpec((640, 320), lambda i: (0, 0)),
      ],
      out_specs=[
          pl.BlockSpec((RB, 128), lambda i: (i, 0)),
          pl.BlockSpec((RB, 128), lambda i: (i, 0)),
          pl.BlockSpec((RB, 128), lambda i: (i, 0)),
      ],
      out_shape=[_SDS((NP, 128), _f32)] * 3,
  )(p0, p1, g1, dinv, w2, b2, w3)


def _k4_body(oa_ref, ob_ref, q0_ref, q1_ref, gc_ref, dinv_ref, b3_ref,
             batch_ref, sums_ref, cnt_ref):
  i = pl.program_id(0)
  d = dinv_ref[...]
  sc2 = q0_ref[...] + q1_ref[...] - gc_ref[...]
  s2 = jnp.concatenate([oa_ref[...], ob_ref[:, :64], sc2], axis=1)
  x3 = jnp.maximum(s2 * d + b3_ref[...], 0.0)
  seg = lax.broadcasted_iota(jnp.int32, (RB, NB), 1)
  oh = (batch_ref[...] == seg).astype(_f32)
  psum = lax.dot_general(oh, x3, (((0,), (0,)), ((), ())),
                         preferred_element_type=_f32)
  pcnt = lax.dot_general(oh, jnp.ones((RB, 128), _f32), (((0,), (0,)), ((), ())),
                         preferred_element_type=_f32)

  @pl.when(i == 0)
  def _():
    sums_ref[...] = psum
    cnt_ref[...] = pcnt

  @pl.when(i > 0)
  def _():
    sums_ref[...] += psum
    cnt_ref[...] += pcnt


def _k4(oa, ob, q0, q1, g2c, dinv, b3, batch1):
  return pl.pallas_call(
      _k4_body,
      grid=(NBLK,),
      in_specs=[
          pl.BlockSpec((RB, 128), lambda i: (i, 0)),
          pl.BlockSpec((RB, 128), lambda i: (i, 0)),
          pl.BlockSpec((RB, 128), lambda i: (i, 0)),
          pl.BlockSpec((RB, 128), lambda i: (i, 0)),
          pl.BlockSpec((RB, 128), lambda i: (i, 0)),
          pl.BlockSpec((RB, 1), lambda i: (i, 0)),
          pl.BlockSpec((1, 320), lambda i: (0, 0)),
          pl.BlockSpec((RB, 1), lambda i: (i, 0)),
      ],
      out_specs=[
          pl.BlockSpec((NB, 320), lambda i: (0, 0)),
          pl.BlockSpec((NB, 128), lambda i: (0, 0)),
      ],
      out_shape=[_SDS((NB, 320), _f32), _SDS((NB, 128), _f32)],
      compiler_params=pltpu.CompilerParams(dimension_semantics=("arbitrary",)),
  )(oa, ob, q0, q1, g2c, dinv, b3, batch1)


def _k5_body(sums_ref, cnt_ref, fp_ref, wg1_ref, bg1_ref, wg2_ref, bg2_ref,
             wf1a_ref, wf1b_ref, bf1_ref, wf2_ref, bf2_ref, wo_ref, bo_ref,
             out_ref):
  cnt = jnp.maximum(cnt_ref[:, :1], 1.0)
  mean = sums_ref[...] / cnt
  z1 = jnp.maximum(jnp.dot(mean, wg1_ref[...], preferred_element_type=_f32)
                   + bg1_ref[...], 0.0)
  z2 = jnp.dot(z1, wg2_ref[...], preferred_element_type=_f32) + bg2_ref[...]
  t = (jnp.dot(fp_ref[...], wf1a_ref[...], preferred_element_type=_f32)
       + jnp.dot(z2, wf1b_ref[...], preferred_element_type=_f32)
       + bf1_ref[...])
  t = jnp.maximum(t, 0.0)
  t2 = jnp.maximum(jnp.dot(t, wf2_ref[...], preferred_element_type=_f32)
                   + bf2_ref[...], 0.0)
  out_ref[...] = (jnp.dot(t2, wo_ref[...], preferred_element_type=_f32)
                  + bo_ref[...])


def _k5(sums, cnt, fp_pad, wg1, bg1, wg2, bg2, wf1a, wf1b, bf1, wf2, bf2, wo, bo):
  return pl.pallas_call(
      _k5_body,
      out_shape=_SDS((NB, 1), _f32),
  )(sums, cnt, fp_pad, wg1, bg1, wg2, bg2, wf1a, wf1b, bf1, wf2, bf2, wo, bo)


def kernel(x, edge_index, batch, data_fp, W1, b1, W2, b2, W3, b3,
           Wg1, bg1, Wg2, bg2, Wf1, bf1, Wf2, bf2, Wo, bo):
  pad_idx = (N + (jnp.arange(EP - E, dtype=jnp.int32) % (NP - N))).astype(jnp.int32)
  src2 = jnp.concatenate([edge_index[0], pad_idx]).reshape(ER, W)
  dst2 = jnp.concatenate([edge_index[1], pad_idx]).reshape(ER, W)
  x_pad = jnp.pad(x, ((0, NP - N), (0, 0)))
  batch1 = jnp.pad(batch, (0, NP - N), constant_values=NB).reshape(NP, 1)
  zeros_np = jnp.zeros((NP,), _f32)
  fp_pad = jnp.pad(data_fp.astype(_f32), ((0, 0), (0, 63)))
  wf1a = jnp.pad(Wf1[:65], ((0, 63), (0, 0)))
  wf1b = Wf1[65:]
  b1r = b1.reshape(1, 128)
  b2r = b2.reshape(1, 640)
  b3r = b3.reshape(1, 320)
  bg1r, bg2r = bg1.reshape(1, 1024), bg2.reshape(1, 128)
  bf1r, bf2r = bf1.reshape(1, 1024), bf2.reshape(1, 512)
  bor = bo.reshape(1, 1)

  dega, degb = _deg_kernel()(dst2, zeros_np)
  dinv, g0 = _k1(dega.reshape(NP, 1), degb.reshape(NP, 1), x_pad)

  p0, p1 = _agg_split_kernel()(g0, src2, dst2)
  g1 = _k2(p0, p1, g0, dinv, W1, b1r)

  p0, p1 = _agg_split_kernel()(g1, src2, dst2)
  g2a, g2b, g2c = _k3(p0, p1, g1, dinv, W2, b2r, W3)

  oa, ob = _agg_dual_kernel()(g2a, g2b, src2, dst2)
  q0, q1 = _agg_split_kernel()(g2c, src2, dst2)
  sums, cnt = _k4(oa, ob, q0, q1, g2c, dinv, b3r, batch1)

  return _k5(sums, cnt, fp_pad, Wg1, bg1r, Wg2, bg2r,
             wf1a, wf1b, bf1r, Wf2, bf2r, Wo, bor)


# speedup vs baseline: 27.2789x; 1.0078x over previous
import functools

import jax
import jax.numpy as jnp
from jax import lax
from jax.experimental import pallas as pl
from jax.experimental.pallas import tpu as pltpu
from jax.experimental.pallas import tpu_sc as plsc

N = 10000
NP = 10240
E = 320000
EP = 327680
ER = EP // 64
RPT = ER // 16
RPT_H = ER // 32
NB = 32
RB = 1024
NBLK = NP // RB

_f32 = jnp.float32
_SDS = jax.ShapeDtypeStruct


def _sc_mesh():
  return plsc.VectorSubcoreMesh(core_axis_name="c", subcore_axis_name="s")


def _deg_body(dst3, zeros_np, out_a, out_b, sdeg, dstbuf, ones_v):
  c = lax.axis_index("c")
  s = lax.axis_index("s")
  for i in range(4):
    ones_v[pl.ds(i * 16, 16)] = jnp.full((16,), 1.0, _f32)
  pltpu.sync_copy(zeros_np.at[pl.ds(s * 640, 640)], sdeg.at[pl.ds(s * 640, 640)])
  pltpu.sync_copy(dst3.at[pl.ds(c * (RPT_H * 16) + s * RPT_H, RPT_H)], dstbuf)
  plsc.subcore_barrier()

  def step(j, carry):
    pltpu.sync_copy(ones_v, sdeg.at[dstbuf.at[j]], add=True)
    return carry

  lax.fori_loop(0, RPT_H, step, 0)
  plsc.subcore_barrier()

  @pl.when(c == 0)
  def _():
    pltpu.sync_copy(sdeg.at[pl.ds(s * 640, 640)], out_a.at[pl.ds(s * 640, 640)])

  @pl.when(c == 1)
  def _():
    pltpu.sync_copy(sdeg.at[pl.ds(s * 640, 640)], out_b.at[pl.ds(s * 640, 640)])


@functools.cache
def _deg_kernel():
  return pl.kernel(
      _deg_body,
      out_type=(_SDS((NP,), _f32), _SDS((NP,), _f32)),
      mesh=_sc_mesh(),
      scratch_types=[
          pltpu.VMEM_SHARED((NP,), _f32),
          pltpu.VMEM((RPT_H, W), jnp.int32),
          pltpu.VMEM((W,), _f32),
      ],
  )


W = 64
IDXB = 40
NBUF = 4


def _edge_pass(g_ref, acc, src3, dst3, base, nrows,
               srcblk, dstblk, bufs, gsems, ssems):
  def blk(bi, carry):
    off = base + bi * IDXB
    pltpu.sync_copy(src3.at[pl.ds(off, IDXB)], srcblk)
    pltpu.sync_copy(dst3.at[pl.ds(off, IDXB)], dstblk)
    for b in range(2):
      @pl.when(bi > 0)
      def _():
        pltpu.make_async_copy(bufs[b], acc.at[dstblk.at[0]], ssems[b]).wait()
      pltpu.async_copy(g_ref.at[srcblk.at[b]], bufs[b], gsems[b])

    def grp(gi, c2):
      for b in range(NBUF):
        j = NBUF * gi + b
        jg = bi * IDXB + j
        b2 = (b + 2) % NBUF

        @pl.when(j + 2 < IDXB)
        def _():
          @pl.when(jg >= 2)
          def _():
            pltpu.make_async_copy(bufs[b2], acc.at[dstblk.at[j]],
                                  ssems[b2]).wait()
          pltpu.async_copy(g_ref.at[srcblk.at[j + 2]], bufs[b2], gsems[b2])

        pltpu.make_async_copy(g_ref.at[srcblk.at[j]], bufs[b], gsems[b]).wait()
        pltpu.async_copy(bufs[b], acc.at[dstblk.at[j]], ssems[b], add=True)
      return c2

    lax.fori_loop(0, IDXB // NBUF, grp, 0)
    return carry

  lax.fori_loop(0, nrows // IDXB, blk, 0)
  for b in range(NBUF):
    pltpu.make_async_copy(bufs[b], acc.at[dstblk.at[0]], ssems[b]).wait()


def _agg_split_body(g, src3, dst3, out0, out1,
                    acc, srcblk, dstblk,
                    b0, b1, b2, b3, ga0, ga1, ga2, ga3, sa0, sa1, sa2, sa3):
  c = lax.axis_index("c")
  s = lax.axis_index("s")
  pltpu.sync_copy(g.at[pl.ds(s * 640, 640)], acc.at[pl.ds(s * 640, 640)])
  plsc.subcore_barrier()
  _edge_pass(g, acc, src3, dst3, c * (RPT_H * 16) + s * RPT_H, RPT_H,
             srcblk, dstblk, [b0, b1, b2, b3],
             [ga0, ga1, ga2, ga3], [sa0, sa1, sa2, sa3])
  plsc.subcore_barrier()

  @pl.when(c == 0)
  def _():
    pltpu.sync_copy(acc.at[pl.ds(s * 640, 640)], out0.at[pl.ds(s * 640, 640)])

  @pl.when(c == 1)
  def _():
    pltpu.sync_copy(acc.at[pl.ds(s * 640, 640)], out1.at[pl.ds(s * 640, 640)])


def _agg_dual_body(g_a, g_b, src3, dst3, out_a, out_b,
                   acc, srcblk, dstblk,
                   b0, b1, b2, b3, ga0, ga1, ga2, ga3, sa0, sa1, sa2, sa3):
  c = lax.axis_index("c")
  s = lax.axis_index("s")

  def run(g_ref, out_ref):
    pltpu.sync_copy(g_ref.at[pl.ds(s * 640, 640)], acc.at[pl.ds(s * 640, 640)])
    plsc.subcore_barrier()
    _edge_pass(g_ref, acc, src3, dst3, s * RPT, RPT,
               srcblk, dstblk, [b0, b1, b2, b3],
               [ga0, ga1, ga2, ga3], [sa0, sa1, sa2, sa3])
    plsc.subcore_barrier()
    pltpu.sync_copy(acc.at[pl.ds(s * 640, 640)], out_ref.at[pl.ds(s * 640, 640)])

  @pl.when(c == 0)
  def _():
    run(g_a, out_a)

  @pl.when(c == 1)
  def _():
    run(g_b, out_b)


def _agg_scratch():
  return ([pltpu.VMEM_SHARED((NP, 128), _f32),
           pltpu.VMEM((IDXB, W), jnp.int32),
           pltpu.VMEM((IDXB, W), jnp.int32)]
          + [pltpu.VMEM((W, 128), _f32)] * NBUF
          + [pltpu.SemaphoreType.DMA] * (2 * NBUF))


@functools.cache
def _agg_split_kernel():
  return pl.kernel(
      _agg_split_body,
      out_type=(_SDS((NP, 128), _f32), _SDS((NP, 128), _f32)),
      mesh=_sc_mesh(),
      scratch_types=_agg_scratch(),
  )


@functools.cache
def _agg_dual_kernel():
  return pl.kernel(
      _agg_dual_body,
      out_type=(_SDS((NP, 128), _f32), _SDS((NP, 128), _f32)),
      mesh=_sc_mesh(),
      scratch_types=_agg_scratch(),
  )


def _row_mask(i):
  rid = i * RB + lax.broadcasted_iota(jnp.int32, (RB, 1), 0)
  return (rid < N).astype(_f32)


def _k1_body(dega_ref, degb_ref, x_ref, dinv_ref, g_ref):
  i = pl.program_id(0)
  deg = dega_ref[...] + degb_ref[...] + 1.0
  dinv = lax.rsqrt(jnp.maximum(deg, 1.0))
  dinv_ref[...] = dinv
  g_ref[...] = x_ref[...] * dinv * _row_mask(i)


def _k1(dega, degb, x_pad):
  return pl.pallas_call(
      _k1_body,
      grid=(NBLK,),
      in_specs=[
          pl.BlockSpec((RB, 1), lambda i: (i, 0)),
          pl.BlockSpec((RB, 1), lambda i: (i, 0)),
          pl.BlockSpec((RB, 128), lambda i: (i, 0)),
      ],
      out_specs=[
          pl.BlockSpec((RB, 1), lambda i: (i, 0)),
          pl.BlockSpec((RB, 128), lambda i: (i, 0)),
      ],
      out_shape=[_SDS((NP, 1), _f32), _SDS((NP, 128), _f32)],
  )(dega, degb, x_pad)


def _k2_body(p0_ref, p1_ref, g_ref, dinv_ref, w_ref, b_ref, out_ref):
  i = pl.program_id(0)
  d = dinv_ref[...]
  sm = (p0_ref[...] + p1_ref[...] - g_ref[...]) * d
  t = jnp.dot(sm, w_ref[...], preferred_element_type=_f32) + b_ref[...]
  out_ref[...] = jnp.maximum(t, 0.0) * d * _row_mask(i)


def _k2(p0, p1, g0, dinv, w1, b1):
  return pl.pallas_call(
      _k2_body,
      grid=(NBLK,),
      in_specs=[
          pl.BlockSpec((RB, 128), lambda i: (i, 0)),
          pl.BlockSpec((RB, 128), lambda i: (i, 0)),
          pl.BlockSpec((RB, 128), lambda i: (i, 0)),
          pl.BlockSpec((RB, 1), lambda i: (i, 0)),
          pl.BlockSpec((128, 128), lambda i: (0, 0)),
          pl.BlockSpec((1, 128), lambda i: (0, 0)),
      ],
      out_specs=pl.BlockSpec((RB, 128), lambda i: (i, 0)),
      out_shape=_SDS((NP, 128), _f32),
  )(p0, p1, g0, dinv, w1, b1)


def _k3_body(p0_ref, p1_ref, g_ref, dinv_ref, w2_ref, b2_ref, w3_ref,
             ga_ref, gb_ref, gc_ref):
  i = pl.program_id(0)
  d = dinv_ref[...]
  sm = (p0_ref[...] + p1_ref[...] - g_ref[...]) * d
  u = jnp.dot(sm, w2_ref[...], preferred_element_type=_f32) + b2_ref[...]
  u = jnp.maximum(u, 0.0)
  h = jnp.dot(u, w3_ref[...], preferred_element_type=_f32)
  g = h * d * _row_mask(i)
  ga_ref[...] = g[:, :128]
  gb_ref[...] = g[:, 128:256]
  gc_ref[...] = g[:, 192:]


def _k3(p0, p1, g1, dinv, w2, b2, w3):
  return pl.pallas_call(
      _k3_body,
      grid=(NBLK,),
      in_specs=[
          pl.BlockSpec((RB, 128), lambda i: (i, 0)),
          pl.BlockSpec((RB, 128), lambda i: (i, 0)),
          pl.BlockSpec((RB, 128), lambda i: (i, 0)),
          pl.BlockSpec((RB, 1), lambda i: (i, 0)),
          pl.BlockSpec((128, 640), lambda i: (0, 0)),
          pl.BlockSpec((1, 640), lambda i: (0, 0)),
          pl.BlockSpec((640, 320), lambda i: (0, 0)),
      ],
      out_specs=[
          pl.BlockSpec((RB, 128), lambda i: (i, 0)),
          pl.BlockSpec((RB, 128), lambda i: (i, 0)),
          pl.BlockSpec((RB, 128), lambda i: (i, 0)),
      ],
      out_shape=[_SDS((NP, 128), _f32)] * 3,
  )(p0, p1, g1, dinv, w2, b2, w3)


def _k4_body(oa_ref, ob_ref, q0_ref, q1_ref, gc_ref, dinv_ref, b3_ref,
             batch_ref, sums_ref, cnt_ref):
  i = pl.program_id(0)
  d = dinv_ref[...]
  sc2 = q0_ref[...] + q1_ref[...] - gc_ref[...]
  s2 = jnp.concatenate([oa_ref[...], ob_ref[:, :64], sc2], axis=1)
  x3 = jnp.maximum(s2 * d + b3_ref[...], 0.0)
  seg = lax.broadcasted_iota(jnp.int32, (RB, NB), 1)
  oh = (batch_ref[...] == seg).astype(_f32)
  psum = lax.dot_general(oh, x3, (((0,), (0,)), ((), ())),
                         preferred_element_type=_f32)
  pcnt = lax.dot_general(oh, jnp.ones((RB, 128), _f32), (((0,), (0,)), ((), ())),
                         preferred_element_type=_f32)

  @pl.when(i == 0)
  def _():
    sums_ref[...] = psum
    cnt_ref[...] = pcnt

  @pl.when(i > 0)
  def _():
    sums_ref[...] += psum
    cnt_ref[...] += pcnt


def _k4(oa, ob, q0, q1, g2c, dinv, b3, batch1):
  return pl.pallas_call(
      _k4_body,
      grid=(NBLK,),
      in_specs=[
          pl.BlockSpec((RB, 128), lambda i: (i, 0)),
          pl.BlockSpec((RB, 128), lambda i: (i, 0)),
          pl.BlockSpec((RB, 128), lambda i: (i, 0)),
          pl.BlockSpec((RB, 128), lambda i: (i, 0)),
          pl.BlockSpec((RB, 128), lambda i: (i, 0)),
          pl.BlockSpec((RB, 1), lambda i: (i, 0)),
          pl.BlockSpec((1, 320), lambda i: (0, 0)),
          pl.BlockSpec((RB, 1), lambda i: (i, 0)),
      ],
      out_specs=[
          pl.BlockSpec((NB, 320), lambda i: (0, 0)),
          pl.BlockSpec((NB, 128), lambda i: (0, 0)),
      ],
      out_shape=[_SDS((NB, 320), _f32), _SDS((NB, 128), _f32)],
      compiler_params=pltpu.CompilerParams(dimension_semantics=("arbitrary",)),
  )(oa, ob, q0, q1, g2c, dinv, b3, batch1)


def _k5_body(sums_ref, cnt_ref, fp_ref, wg1_ref, bg1_ref, wg2_ref, bg2_ref,
             wf1a_ref, wf1b_ref, bf1_ref, wf2_ref, bf2_ref, wo_ref, bo_ref,
             out_ref):
  cnt = jnp.maximum(cnt_ref[:, :1], 1.0)
  mean = sums_ref[...] / cnt
  z1 = jnp.maximum(jnp.dot(mean, wg1_ref[...], preferred_element_type=_f32)
                   + bg1_ref[...], 0.0)
  z2 = jnp.dot(z1, wg2_ref[...], preferred_element_type=_f32) + bg2_ref[...]
  t = (jnp.dot(fp_ref[...], wf1a_ref[...], preferred_element_type=_f32)
       + jnp.dot(z2, wf1b_ref[...], preferred_element_type=_f32)
       + bf1_ref[...])
  t = jnp.maximum(t, 0.0)
  t2 = jnp.maximum(jnp.dot(t, wf2_ref[...], preferred_element_type=_f32)
                   + bf2_ref[...], 0.0)
  out_ref[...] = (jnp.dot(t2, wo_ref[...], preferred_element_type=_f32)
                  + bo_ref[...])


def _k5(sums, cnt, fp_pad, wg1, bg1, wg2, bg2, wf1a, wf1b, bf1, wf2, bf2, wo, bo):
  return pl.pallas_call(
      _k5_body,
      out_shape=_SDS((NB, 1), _f32),
  )(sums, cnt, fp_pad, wg1, bg1, wg2, bg2, wf1a, wf1b, bf1, wf2, bf2, wo, bo)


def kernel(x, edge_index, batch, data_fp, W1, b1, W2, b2, W3, b3,
           Wg1, bg1, Wg2, bg2, Wf1, bf1, Wf2, bf2, Wo, bo):
  pad_idx = (N + (jnp.arange(EP - E, dtype=jnp.int32) % (NP - N))).astype(jnp.int32)
  src2 = jnp.concatenate([edge_index[0], pad_idx]).reshape(ER, W)
  dst2 = jnp.concatenate([edge_index[1], pad_idx]).reshape(ER, W)
  x_pad = jnp.pad(x, ((0, NP - N), (0, 0)))
  batch1 = jnp.pad(batch, (0, NP - N), constant_values=NB).reshape(NP, 1)
  zeros_np = jnp.zeros((NP,), _f32)
  fp_pad = jnp.pad(data_fp.astype(_f32), ((0, 0), (0, 63)))
  wf1a = jnp.pad(Wf1[:65], ((0, 63), (0, 0)))
  wf1b = Wf1[65:]
  b1r = b1.reshape(1, 128)
  b2r = b2.reshape(1, 640)
  b3r = b3.reshape(1, 320)
  bg1r, bg2r = bg1.reshape(1, 1024), bg2.reshape(1, 128)
  bf1r, bf2r = bf1.reshape(1, 1024), bf2.reshape(1, 512)
  bor = bo.reshape(1, 1)

  dega, degb = _deg_kernel()(dst2, zeros_np)
  dinv, g0 = _k1(dega.reshape(NP, 1), degb.reshape(NP, 1), x_pad)

  p0, p1 = _agg_split_kernel()(g0, src2, dst2)
  g1 = _k2(p0, p1, g0, dinv, W1, b1r)

  p0, p1 = _agg_split_kernel()(g1, src2, dst2)
  g2a, g2b, g2c = _k3(p0, p1, g1, dinv, W2, b2r, W3)

  oa, ob = _agg_dual_kernel()(g2a, g2b, src2, dst2)
  q0, q1 = _agg_split_kernel()(g2c, src2, dst2)
  sums, cnt = _k4(oa, ob, q0, q1, g2c, dinv, b3r, batch1)

  return _k5(sums, cnt, fp_pad, Wg1, bg1r, Wg2, bg2r,
             wf1a, wf1b, bf1r, Wf2, bf2r, Wo, bor)


# speedup vs baseline: 27.7175x; 1.0161x over previous
import functools

import jax
import jax.numpy as jnp
from jax import lax
from jax.experimental import pallas as pl
from jax.experimental.pallas import tpu as pltpu
from jax.experimental.pallas import tpu_sc as plsc

N = 10000
NP = 10240
E = 320000
EP = 327680
ER = EP // 64
RPT = ER // 16
RPT_H = ER // 32
NB = 32
RB = 1024
NBLK = NP // RB

_f32 = jnp.float32
_SDS = jax.ShapeDtypeStruct


def _sc_mesh():
  return plsc.VectorSubcoreMesh(core_axis_name="c", subcore_axis_name="s")


def _deg_body(dst3, zeros_np, out_a, out_b, sdeg, dstbuf, ones_v, dsem):
  c = lax.axis_index("c")
  s = lax.axis_index("s")
  for i in range(4):
    ones_v[pl.ds(i * 16, 16)] = jnp.full((16,), 1.0, _f32)
  pltpu.sync_copy(zeros_np.at[pl.ds(s * 640, 640)], sdeg.at[pl.ds(s * 640, 640)])
  pltpu.sync_copy(dst3.at[pl.ds(c * (RPT_H * 16) + s * RPT_H, RPT_H)], dstbuf)
  plsc.subcore_barrier()

  def step(j, carry):
    pltpu.async_copy(ones_v, sdeg.at[dstbuf.at[j]], dsem, add=True)
    return carry

  lax.fori_loop(0, RPT_H, step, 0)

  def drain(j, carry):
    pltpu.make_async_copy(ones_v, sdeg.at[dstbuf.at[0]], dsem).wait()
    return carry

  lax.fori_loop(0, RPT_H, drain, 0)
  plsc.subcore_barrier()

  @pl.when(c == 0)
  def _():
    pltpu.sync_copy(sdeg.at[pl.ds(s * 640, 640)], out_a.at[pl.ds(s * 640, 640)])

  @pl.when(c == 1)
  def _():
    pltpu.sync_copy(sdeg.at[pl.ds(s * 640, 640)], out_b.at[pl.ds(s * 640, 640)])


@functools.cache
def _deg_kernel():
  return pl.kernel(
      _deg_body,
      out_type=(_SDS((NP,), _f32), _SDS((NP,), _f32)),
      mesh=_sc_mesh(),
      scratch_types=[
          pltpu.VMEM_SHARED((NP,), _f32),
          pltpu.VMEM((RPT_H, W), jnp.int32),
          pltpu.VMEM((W,), _f32),
          pltpu.SemaphoreType.DMA,
      ],
  )


W = 64
IDXB = 40
NBUF = 4


def _edge_pass(g_ref, acc, src3, dst3, base, nrows,
               srcblk, dstblk, bufs, gsems, ssems):
  def blk(bi, carry):
    off = base + bi * IDXB
    pltpu.sync_copy(src3.at[pl.ds(off, IDXB)], srcblk)
    pltpu.sync_copy(dst3.at[pl.ds(off, IDXB)], dstblk)
    for b in range(2):
      @pl.when(bi > 0)
      def _():
        pltpu.make_async_copy(bufs[b], acc.at[dstblk.at[0]], ssems[b]).wait()
      pltpu.async_copy(g_ref.at[srcblk.at[b]], bufs[b], gsems[b])

    def grp(gi, c2):
      for b in range(NBUF):
        j = NBUF * gi + b
        jg = bi * IDXB + j
        b2 = (b + 2) % NBUF

        @pl.when(j + 2 < IDXB)
        def _():
          @pl.when(jg >= 2)
          def _():
            pltpu.make_async_copy(bufs[b2], acc.at[dstblk.at[j]],
                                  ssems[b2]).wait()
          pltpu.async_copy(g_ref.at[srcblk.at[j + 2]], bufs[b2], gsems[b2])

        pltpu.make_async_copy(g_ref.at[srcblk.at[j]], bufs[b], gsems[b]).wait()
        pltpu.async_copy(bufs[b], acc.at[dstblk.at[j]], ssems[b], add=True)
      return c2

    lax.fori_loop(0, IDXB // NBUF, grp, 0)
    return carry

  lax.fori_loop(0, nrows // IDXB, blk, 0)
  for b in range(NBUF):
    pltpu.make_async_copy(bufs[b], acc.at[dstblk.at[0]], ssems[b]).wait()


def _agg_split_body(g, src3, dst3, out0, out1,
                    acc, srcblk, dstblk,
                    b0, b1, b2, b3, ga0, ga1, ga2, ga3, sa0, sa1, sa2, sa3):
  c = lax.axis_index("c")
  s = lax.axis_index("s")
  pltpu.sync_copy(g.at[pl.ds(s * 640, 640)], acc.at[pl.ds(s * 640, 640)])
  plsc.subcore_barrier()
  _edge_pass(g, acc, src3, dst3, c * (RPT_H * 16) + s * RPT_H, RPT_H,
             srcblk, dstblk, [b0, b1, b2, b3],
             [ga0, ga1, ga2, ga3], [sa0, sa1, sa2, sa3])
  plsc.subcore_barrier()

  @pl.when(c == 0)
  def _():
    pltpu.sync_copy(acc.at[pl.ds(s * 640, 640)], out0.at[pl.ds(s * 640, 640)])

  @pl.when(c == 1)
  def _():
    pltpu.sync_copy(acc.at[pl.ds(s * 640, 640)], out1.at[pl.ds(s * 640, 640)])


def _agg_dual_body(g_a, g_b, src3, dst3, out_a, out_b,
                   acc, srcblk, dstblk,
                   b0, b1, b2, b3, ga0, ga1, ga2, ga3, sa0, sa1, sa2, sa3):
  c = lax.axis_index("c")
  s = lax.axis_index("s")

  def run(g_ref, out_ref):
    pltpu.sync_copy(g_ref.at[pl.ds(s * 640, 640)], acc.at[pl.ds(s * 640, 640)])
    plsc.subcore_barrier()
    _edge_pass(g_ref, acc, src3, dst3, s * RPT, RPT,
               srcblk, dstblk, [b0, b1, b2, b3],
               [ga0, ga1, ga2, ga3], [sa0, sa1, sa2, sa3])
    plsc.subcore_barrier()
    pltpu.sync_copy(acc.at[pl.ds(s * 640, 640)], out_ref.at[pl.ds(s * 640, 640)])

  @pl.when(c == 0)
  def _():
    run(g_a, out_a)

  @pl.when(c == 1)
  def _():
    run(g_b, out_b)


def _agg_scratch():
  return ([pltpu.VMEM_SHARED((NP, 128), _f32),
           pltpu.VMEM((IDXB, W), jnp.int32),
           pltpu.VMEM((IDXB, W), jnp.int32)]
          + [pltpu.VMEM((W, 128), _f32)] * NBUF
          + [pltpu.SemaphoreType.DMA] * (2 * NBUF))


@functools.cache
def _agg_split_kernel():
  return pl.kernel(
      _agg_split_body,
      out_type=(_SDS((NP, 128), _f32), _SDS((NP, 128), _f32)),
      mesh=_sc_mesh(),
      scratch_types=_agg_scratch(),
  )


@functools.cache
def _agg_dual_kernel():
  return pl.kernel(
      _agg_dual_body,
      out_type=(_SDS((NP, 128), _f32), _SDS((NP, 128), _f32)),
      mesh=_sc_mesh(),
      scratch_types=_agg_scratch(),
  )


def _row_mask(i):
  rid = i * RB + lax.broadcasted_iota(jnp.int32, (RB, 1), 0)
  return (rid < N).astype(_f32)


def _k1_body(dega_ref, degb_ref, x_ref, dinv_ref, g_ref):
  i = pl.program_id(0)
  deg = dega_ref[...] + degb_ref[...] + 1.0
  dinv = lax.rsqrt(jnp.maximum(deg, 1.0))
  dinv_ref[...] = dinv
  g_ref[...] = x_ref[...] * dinv * _row_mask(i)


def _k1(dega, degb, x_pad):
  return pl.pallas_call(
      _k1_body,
      grid=(NBLK,),
      in_specs=[
          pl.BlockSpec((RB, 1), lambda i: (i, 0)),
          pl.BlockSpec((RB, 1), lambda i: (i, 0)),
          pl.BlockSpec((RB, 128), lambda i: (i, 0)),
      ],
      out_specs=[
          pl.BlockSpec((RB, 1), lambda i: (i, 0)),
          pl.BlockSpec((RB, 128), lambda i: (i, 0)),
      ],
      out_shape=[_SDS((NP, 1), _f32), _SDS((NP, 128), _f32)],
  )(dega, degb, x_pad)


def _k2_body(p0_ref, p1_ref, g_ref, dinv_ref, w_ref, b_ref, out_ref):
  i = pl.program_id(0)
  d = dinv_ref[...]
  sm = (p0_ref[...] + p1_ref[...] - g_ref[...]) * d
  t = jnp.dot(sm, w_ref[...], preferred_element_type=_f32) + b_ref[...]
  out_ref[...] = jnp.maximum(t, 0.0) * d * _row_mask(i)


def _k2(p0, p1, g0, dinv, w1, b1):
  return pl.pallas_call(
      _k2_body,
      grid=(NBLK,),
      in_specs=[
          pl.BlockSpec((RB, 128), lambda i: (i, 0)),
          pl.BlockSpec((RB, 128), lambda i: (i, 0)),
          pl.BlockSpec((RB, 128), lambda i: (i, 0)),
          pl.BlockSpec((RB, 1), lambda i: (i, 0)),
          pl.BlockSpec((128, 128), lambda i: (0, 0)),
          pl.BlockSpec((1, 128), lambda i: (0, 0)),
      ],
      out_specs=pl.BlockSpec((RB, 128), lambda i: (i, 0)),
      out_shape=_SDS((NP, 128), _f32),
  )(p0, p1, g0, dinv, w1, b1)


def _k3_body(p0_ref, p1_ref, g_ref, dinv_ref, w2_ref, b2_ref, w3_ref,
             ga_ref, gb_ref, gc_ref):
  i = pl.program_id(0)
  d = dinv_ref[...]
  sm = (p0_ref[...] + p1_ref[...] - g_ref[...]) * d
  u = jnp.dot(sm, w2_ref[...], preferred_element_type=_f32) + b2_ref[...]
  u = jnp.maximum(u, 0.0)
  h = jnp.dot(u, w3_ref[...], preferred_element_type=_f32)
  g = h * d * _row_mask(i)
  ga_ref[...] = g[:, :128]
  gb_ref[...] = g[:, 128:256]
  gc_ref[...] = g[:, 192:]


def _k3(p0, p1, g1, dinv, w2, b2, w3):
  return pl.pallas_call(
      _k3_body,
      grid=(NBLK,),
      in_specs=[
          pl.BlockSpec((RB, 128), lambda i: (i, 0)),
          pl.BlockSpec((RB, 128), lambda i: (i, 0)),
          pl.BlockSpec((RB, 128), lambda i: (i, 0)),
          pl.BlockSpec((RB, 1), lambda i: (i, 0)),
          pl.BlockSpec((128, 640), lambda i: (0, 0)),
          pl.BlockSpec((1, 640), lambda i: (0, 0)),
          pl.BlockSpec((640, 320), lambda i: (0, 0)),
      ],
      out_specs=[
          pl.BlockSpec((RB, 128), lambda i: (i, 0)),
          pl.BlockSpec((RB, 128), lambda i: (i, 0)),
          pl.BlockSpec((RB, 128), lambda i: (i, 0)),
      ],
      out_shape=[_SDS((NP, 128), _f32)] * 3,
  )(p0, p1, g1, dinv, w2, b2, w3)


def _k45_body(oa_ref, ob_ref, q0_ref, q1_ref, gc_ref, dinv_ref, b3_ref,
              batch_ref, fp_ref, wg1_ref, bg1_ref, wg2_ref, bg2_ref,
              wf1a_ref, wf1b_ref, bf1_ref, wf2_ref, bf2_ref, wo_ref, bo_ref,
              out_ref, sums_ref, cnt_ref):
  i = pl.program_id(0)

  @pl.when(i < NBLK)
  def _():
    d = dinv_ref[...]
    sc2 = q0_ref[...] + q1_ref[...] - gc_ref[...]
    s2 = jnp.concatenate([oa_ref[...], ob_ref[:, :64], sc2], axis=1)
    x3 = jnp.maximum(s2 * d + b3_ref[...], 0.0)
    seg = lax.broadcasted_iota(jnp.int32, (RB, NB), 1)
    oh = (batch_ref[...] == seg).astype(_f32)
    psum = lax.dot_general(oh, x3, (((0,), (0,)), ((), ())),
                           preferred_element_type=_f32)
    pcnt = lax.dot_general(oh, jnp.ones((RB, 128), _f32),
                           (((0,), (0,)), ((), ())),
                           preferred_element_type=_f32)

    @pl.when(i == 0)
    def _():
      sums_ref[...] = psum
      cnt_ref[...] = pcnt

    @pl.when(i > 0)
    def _():
      sums_ref[...] += psum
      cnt_ref[...] += pcnt

  @pl.when(i == NBLK)
  def _():
    cnt = jnp.maximum(cnt_ref[:, :1], 1.0)
    mean = sums_ref[...] / cnt
    z1 = jnp.maximum(jnp.dot(mean, wg1_ref[...], preferred_element_type=_f32)
                     + bg1_ref[...], 0.0)
    z2 = jnp.dot(z1, wg2_ref[...], preferred_element_type=_f32) + bg2_ref[...]
    t = (jnp.dot(fp_ref[...], wf1a_ref[...], preferred_element_type=_f32)
         + jnp.dot(z2, wf1b_ref[...], preferred_element_type=_f32)
         + bf1_ref[...])
    t = jnp.maximum(t, 0.0)
    t2 = jnp.maximum(jnp.dot(t, wf2_ref[...], preferred_element_type=_f32)
                     + bf2_ref[...], 0.0)
    out_ref[...] = (jnp.dot(t2, wo_ref[...], preferred_element_type=_f32)
                    + bo_ref[...])


def _k45(oa, ob, q0, q1, g2c, dinv, b3, batch1, fp_pad,
         wg1, bg1, wg2, bg2, wf1a, wf1b, bf1, wf2, bf2, wo, bo):
  row = lambda i: (jnp.minimum(i, NBLK - 1), 0)
  const = lambda i: (0, 0)
  return pl.pallas_call(
      _k45_body,
      grid=(NBLK + 1,),
      in_specs=[
          pl.BlockSpec((RB, 128), row),
          pl.BlockSpec((RB, 128), row),
          pl.BlockSpec((RB, 128), row),
          pl.BlockSpec((RB, 128), row),
          pl.BlockSpec((RB, 128), row),
          pl.BlockSpec((RB, 1), row),
          pl.BlockSpec((1, 320), const),
          pl.BlockSpec((RB, 1), row),
          pl.BlockSpec((NB, 128), const),
          pl.BlockSpec((320, 1024), const),
          pl.BlockSpec((1, 1024), const),
          pl.BlockSpec((1024, 128), const),
          pl.BlockSpec((1, 128), const),
          pl.BlockSpec((128, 1024), const),
          pl.BlockSpec((128, 1024), const),
          pl.BlockSpec((1, 1024), const),
          pl.BlockSpec((1024, 512), const),
          pl.BlockSpec((1, 512), const),
          pl.BlockSpec((512, 1), const),
          pl.BlockSpec((1, 1), const),
      ],
      out_specs=pl.BlockSpec((NB, 1), const),
      out_shape=_SDS((NB, 1), _f32),
      scratch_shapes=[pltpu.VMEM((NB, 320), _f32), pltpu.VMEM((NB, 128), _f32)],
      compiler_params=pltpu.CompilerParams(dimension_semantics=("arbitrary",)),
  )(oa, ob, q0, q1, g2c, dinv, b3, batch1, fp_pad,
    wg1, bg1, wg2, bg2, wf1a, wf1b, bf1, wf2, bf2, wo, bo)


def kernel(x, edge_index, batch, data_fp, W1, b1, W2, b2, W3, b3,
           Wg1, bg1, Wg2, bg2, Wf1, bf1, Wf2, bf2, Wo, bo):
  pad_idx = (N + (jnp.arange(EP - E, dtype=jnp.int32) % (NP - N))).astype(jnp.int32)
  src2 = jnp.concatenate([edge_index[0], pad_idx]).reshape(ER, W)
  dst2 = jnp.concatenate([edge_index[1], pad_idx]).reshape(ER, W)
  x_pad = jnp.pad(x, ((0, NP - N), (0, 0)))
  batch1 = jnp.pad(batch, (0, NP - N), constant_values=NB).reshape(NP, 1)
  zeros_np = jnp.zeros((NP,), _f32)
  fp_pad = jnp.pad(data_fp.astype(_f32), ((0, 0), (0, 63)))
  wf1a = jnp.pad(Wf1[:65], ((0, 63), (0, 0)))
  wf1b = Wf1[65:]
  b1r = b1.reshape(1, 128)
  b2r = b2.reshape(1, 640)
  b3r = b3.reshape(1, 320)
  bg1r, bg2r = bg1.reshape(1, 1024), bg2.reshape(1, 128)
  bf1r, bf2r = bf1.reshape(1, 1024), bf2.reshape(1, 512)
  bor = bo.reshape(1, 1)

  dega, degb = _deg_kernel()(dst2, zeros_np)
  dinv, g0 = _k1(dega.reshape(NP, 1), degb.reshape(NP, 1), x_pad)

  p0, p1 = _agg_split_kernel()(g0, src2, dst2)
  g1 = _k2(p0, p1, g0, dinv, W1, b1r)

  p0, p1 = _agg_split_kernel()(g1, src2, dst2)
  g2a, g2b, g2c = _k3(p0, p1, g1, dinv, W2, b2r, W3)

  oa, ob = _agg_dual_kernel()(g2a, g2b, src2, dst2)
  q0, q1 = _agg_split_kernel()(g2c, src2, dst2)

  return _k45(oa, ob, q0, q1, g2c, dinv, b3r, batch1, fp_pad,
              Wg1, bg1r, Wg2, bg2r, wf1a, wf1b, bf1r, Wf2, bf2r, Wo, bor)


# speedup vs baseline: 28.1463x; 1.0155x over previous
import functools

import jax
import jax.numpy as jnp
from jax import lax
from jax.experimental import pallas as pl
from jax.experimental.pallas import tpu as pltpu
from jax.experimental.pallas import tpu_sc as plsc

N = 10000
NP = 10240
E = 320000
EP = 327680
W = 128
IDXB = 40
NBUF = 2
LA = 1
ER = EP // W
RPT = ER // 16
RPT_H = ER // 32
NB = 32
RB = 1024
NBLK = NP // RB

_f32 = jnp.float32
_SDS = jax.ShapeDtypeStruct


def _sc_mesh():
  return plsc.VectorSubcoreMesh(core_axis_name="c", subcore_axis_name="s")


def _deg_body(dst3, zeros_np, out_a, out_b, sdeg, dstbuf, ones_v, dsem):
  c = lax.axis_index("c")
  s = lax.axis_index("s")
  for i in range(W // 16):
    ones_v[pl.ds(i * 16, 16)] = jnp.full((16,), 1.0, _f32)
  pltpu.sync_copy(zeros_np.at[pl.ds(s * 640, 640)], sdeg.at[pl.ds(s * 640, 640)])
  pltpu.sync_copy(dst3.at[pl.ds(c * (RPT_H * 16) + s * RPT_H, RPT_H)], dstbuf)
  plsc.subcore_barrier()

  def step(j, carry):
    pltpu.async_copy(ones_v, sdeg.at[dstbuf.at[j]], dsem, add=True)
    return carry

  lax.fori_loop(0, RPT_H, step, 0)

  def drain(j, carry):
    pltpu.make_async_copy(ones_v, sdeg.at[dstbuf.at[0]], dsem).wait()
    return carry

  lax.fori_loop(0, RPT_H, drain, 0)
  plsc.subcore_barrier()

  @pl.when(c == 0)
  def _():
    pltpu.sync_copy(sdeg.at[pl.ds(s * 640, 640)], out_a.at[pl.ds(s * 640, 640)])

  @pl.when(c == 1)
  def _():
    pltpu.sync_copy(sdeg.at[pl.ds(s * 640, 640)], out_b.at[pl.ds(s * 640, 640)])


@functools.cache
def _deg_kernel():
  return pl.kernel(
      _deg_body,
      out_type=(_SDS((NP,), _f32), _SDS((NP,), _f32)),
      mesh=_sc_mesh(),
      scratch_types=[
          pltpu.VMEM_SHARED((NP,), _f32),
          pltpu.VMEM((RPT_H, W), jnp.int32),
          pltpu.VMEM((W,), _f32),
          pltpu.SemaphoreType.DMA,
      ],
  )


def _edge_pass(g_ref, acc, src3, dst3, base, nrows,
               srcblk, dstblk, bufs, gsems, ssems):
  def blk(bi, carry):
    off = base + bi * IDXB
    pltpu.sync_copy(src3.at[pl.ds(off, IDXB)], srcblk)
    pltpu.sync_copy(dst3.at[pl.ds(off, IDXB)], dstblk)
    for b in range(LA):
      @pl.when(bi > 0)
      def _():
        pltpu.make_async_copy(bufs[b], acc.at[dstblk.at[0]], ssems[b]).wait()
      pltpu.async_copy(g_ref.at[srcblk.at[b]], bufs[b], gsems[b])

    def grp(gi, c2):
      for b in range(NBUF):
        j = NBUF * gi + b
        jg = bi * IDXB + j
        b2 = (b + LA) % NBUF

        @pl.when(j + LA < IDXB)
        def _():
          @pl.when(jg >= NBUF - LA)
          def _():
            pltpu.make_async_copy(bufs[b2], acc.at[dstblk.at[j]],
                                  ssems[b2]).wait()
          pltpu.async_copy(g_ref.at[srcblk.at[j + LA]], bufs[b2], gsems[b2])

        pltpu.make_async_copy(g_ref.at[srcblk.at[j]], bufs[b], gsems[b]).wait()
        pltpu.async_copy(bufs[b], acc.at[dstblk.at[j]], ssems[b], add=True)
      return c2

    lax.fori_loop(0, IDXB // NBUF, grp, 0)
    return carry

  lax.fori_loop(0, nrows // IDXB, blk, 0)
  for b in range(NBUF):
    pltpu.make_async_copy(bufs[b], acc.at[dstblk.at[0]], ssems[b]).wait()


def _agg_split_body(g, src3, dst3, out0, out1,
                    acc, srcblk, dstblk, *bufs_sems):
  bufs = list(bufs_sems[:NBUF])
  gsems = list(bufs_sems[NBUF:2 * NBUF])
  ssems = list(bufs_sems[2 * NBUF:])
  c = lax.axis_index("c")
  s = lax.axis_index("s")
  pltpu.sync_copy(g.at[pl.ds(s * 640, 640)], acc.at[pl.ds(s * 640, 640)])
  plsc.subcore_barrier()
  _edge_pass(g, acc, src3, dst3, c * (RPT_H * 16) + s * RPT_H, RPT_H,
             srcblk, dstblk, bufs, gsems, ssems)
  plsc.subcore_barrier()

  @pl.when(c == 0)
  def _():
    pltpu.sync_copy(acc.at[pl.ds(s * 640, 640)], out0.at[pl.ds(s * 640, 640)])

  @pl.when(c == 1)
  def _():
    pltpu.sync_copy(acc.at[pl.ds(s * 640, 640)], out1.at[pl.ds(s * 640, 640)])


def _agg_dual_body(g_a, g_b, src3, dst3, out_a, out_b,
                   acc, srcblk, dstblk, *bufs_sems):
  bufs = list(bufs_sems[:NBUF])
  gsems = list(bufs_sems[NBUF:2 * NBUF])
  ssems = list(bufs_sems[2 * NBUF:])
  c = lax.axis_index("c")
  s = lax.axis_index("s")

  def run(g_ref, out_ref):
    pltpu.sync_copy(g_ref.at[pl.ds(s * 640, 640)], acc.at[pl.ds(s * 640, 640)])
    plsc.subcore_barrier()
    _edge_pass(g_ref, acc, src3, dst3, s * RPT, RPT,
               srcblk, dstblk, bufs, gsems, ssems)
    plsc.subcore_barrier()
    pltpu.sync_copy(acc.at[pl.ds(s * 640, 640)], out_ref.at[pl.ds(s * 640, 640)])

  @pl.when(c == 0)
  def _():
    run(g_a, out_a)

  @pl.when(c == 1)
  def _():
    run(g_b, out_b)


def _agg_scratch():
  return ([pltpu.VMEM_SHARED((NP, 128), _f32),
           pltpu.VMEM((IDXB, W), jnp.int32),
           pltpu.VMEM((IDXB, W), jnp.int32)]
          + [pltpu.VMEM((W, 128), _f32)] * NBUF
          + [pltpu.SemaphoreType.DMA] * (2 * NBUF))


@functools.cache
def _agg_split_kernel():
  return pl.kernel(
      _agg_split_body,
      out_type=(_SDS((NP, 128), _f32), _SDS((NP, 128), _f32)),
      mesh=_sc_mesh(),
      scratch_types=_agg_scratch(),
  )


@functools.cache
def _agg_dual_kernel():
  return pl.kernel(
      _agg_dual_body,
      out_type=(_SDS((NP, 128), _f32), _SDS((NP, 128), _f32)),
      mesh=_sc_mesh(),
      scratch_types=_agg_scratch(),
  )


def _row_mask(i):
  rid = i * RB + lax.broadcasted_iota(jnp.int32, (RB, 1), 0)
  return (rid < N).astype(_f32)


def _k1_body(dega_ref, degb_ref, x_ref, dinv_ref, g_ref):
  i = pl.program_id(0)
  deg = dega_ref[...] + degb_ref[...] + 1.0
  dinv = lax.rsqrt(jnp.maximum(deg, 1.0))
  dinv_ref[...] = dinv
  g_ref[...] = x_ref[...] * dinv * _row_mask(i)


def _k1(dega, degb, x_pad):
  return pl.pallas_call(
      _k1_body,
      grid=(NBLK,),
      in_specs=[
          pl.BlockSpec((RB, 1), lambda i: (i, 0)),
          pl.BlockSpec((RB, 1), lambda i: (i, 0)),
          pl.BlockSpec((RB, 128), lambda i: (i, 0)),
      ],
      out_specs=[
          pl.BlockSpec((RB, 1), lambda i: (i, 0)),
          pl.BlockSpec((RB, 128), lambda i: (i, 0)),
      ],
      out_shape=[_SDS((NP, 1), _f32), _SDS((NP, 128), _f32)],
  )(dega, degb, x_pad)


def _k2_body(p0_ref, p1_ref, g_ref, dinv_ref, w_ref, b_ref, out_ref):
  i = pl.program_id(0)
  d = dinv_ref[...]
  sm = (p0_ref[...] + p1_ref[...] - g_ref[...]) * d
  t = jnp.dot(sm, w_ref[...], preferred_element_type=_f32) + b_ref[...]
  out_ref[...] = jnp.maximum(t, 0.0) * d * _row_mask(i)


def _k2(p0, p1, g0, dinv, w1, b1):
  return pl.pallas_call(
      _k2_body,
      grid=(NBLK,),
      in_specs=[
          pl.BlockSpec((RB, 128), lambda i: (i, 0)),
          pl.BlockSpec((RB, 128), lambda i: (i, 0)),
          pl.BlockSpec((RB, 128), lambda i: (i, 0)),
          pl.BlockSpec((RB, 1), lambda i: (i, 0)),
          pl.BlockSpec((128, 128), lambda i: (0, 0)),
          pl.BlockSpec((1, 128), lambda i: (0, 0)),
      ],
      out_specs=pl.BlockSpec((RB, 128), lambda i: (i, 0)),
      out_shape=_SDS((NP, 128), _f32),
  )(p0, p1, g0, dinv, w1, b1)


def _k3_body(p0_ref, p1_ref, g_ref, dinv_ref, w2_ref, b2_ref, w3_ref,
             ga_ref, gb_ref, gc_ref):
  i = pl.program_id(0)
  d = dinv_ref[...]
  sm = (p0_ref[...] + p1_ref[...] - g_ref[...]) * d
  u = jnp.dot(sm, w2_ref[...], preferred_element_type=_f32) + b2_ref[...]
  u = jnp.maximum(u, 0.0)
  h = jnp.dot(u, w3_ref[...], preferred_element_type=_f32)
  g = h * d * _row_mask(i)
  ga_ref[...] = g[:, :128]
  gb_ref[...] = g[:, 128:256]
  gc_ref[...] = g[:, 192:]


def _k3(p0, p1, g1, dinv, w2, b2, w3):
  return pl.pallas_call(
      _k3_body,
      grid=(NBLK,),
      in_specs=[
          pl.BlockSpec((RB, 128), lambda i: (i, 0)),
          pl.BlockSpec((RB, 128), lambda i: (i, 0)),
          pl.BlockSpec((RB, 128), lambda i: (i, 0)),
          pl.BlockSpec((RB, 1), lambda i: (i, 0)),
          pl.BlockSpec((128, 640), lambda i: (0, 0)),
          pl.BlockSpec((1, 640), lambda i: (0, 0)),
          pl.BlockSpec((640, 320), lambda i: (0, 0)),
      ],
      out_specs=[
          pl.BlockSpec((RB, 128), lambda i: (i, 0)),
          pl.BlockSpec((RB, 128), lambda i: (i, 0)),
          pl.BlockSpec((RB, 128), lambda i: (i, 0)),
      ],
      out_shape=[_SDS((NP, 128), _f32)] * 3,
  )(p0, p1, g1, dinv, w2, b2, w3)


def _k45_body(oa_ref, ob_ref, q0_ref, q1_ref, gc_ref, dinv_ref, b3_ref,
              batch_ref, fp_ref, wg1_ref, bg1_ref, wg2_ref, bg2_ref,
              wf1a_ref, wf1b_ref, bf1_ref, wf2_ref, bf2_ref, wo_ref, bo_ref,
              out_ref, sums_ref, cnt_ref):
  i = pl.program_id(0)

  @pl.when(i < NBLK)
  def _():
    d = dinv_ref[...]
    sc2 = q0_ref[...] + q1_ref[...] - gc_ref[...]
    s2 = jnp.concatenate([oa_ref[...], ob_ref[:, :64], sc2], axis=1)
    x3 = jnp.maximum(s2 * d + b3_ref[...], 0.0)
    seg = lax.broadcasted_iota(jnp.int32, (RB, NB), 1)
    oh = (batch_ref[...] == seg).astype(_f32)
    psum = lax.dot_general(oh, x3, (((0,), (0,)), ((), ())),
                           preferred_element_type=_f32)
    pcnt = lax.dot_general(oh, jnp.ones((RB, 128), _f32),
                           (((0,), (0,)), ((), ())),
                           preferred_element_type=_f32)

    @pl.when(i == 0)
    def _():
      sums_ref[...] = psum
      cnt_ref[...] = pcnt

    @pl.when(i > 0)
    def _():
      sums_ref[...] += psum
      cnt_ref[...] += pcnt

  @pl.when(i == NBLK)
  def _():
    cnt = jnp.maximum(cnt_ref[:, :1], 1.0)
    mean = sums_ref[...] / cnt
    z1 = jnp.maximum(jnp.dot(mean, wg1_ref[...], preferred_element_type=_f32)
                     + bg1_ref[...], 0.0)
    z2 = jnp.dot(z1, wg2_ref[...], preferred_element_type=_f32) + bg2_ref[...]
    t = (jnp.dot(fp_ref[...], wf1a_ref[...], preferred_element_type=_f32)
         + jnp.dot(z2, wf1b_ref[...], preferred_element_type=_f32)
         + bf1_ref[...])
    t = jnp.maximum(t, 0.0)
    t2 = jnp.maximum(jnp.dot(t, wf2_ref[...], preferred_element_type=_f32)
                     + bf2_ref[...], 0.0)
    out_ref[...] = (jnp.dot(t2, wo_ref[...], preferred_element_type=_f32)
                    + bo_ref[...])


def _k45(oa, ob, q0, q1, g2c, dinv, b3, batch1, fp_pad,
         wg1, bg1, wg2, bg2, wf1a, wf1b, bf1, wf2, bf2, wo, bo):
  row = lambda i: (jnp.minimum(i, NBLK - 1), 0)
  const = lambda i: (0, 0)
  return pl.pallas_call(
      _k45_body,
      grid=(NBLK + 1,),
      in_specs=[
          pl.BlockSpec((RB, 128), row),
          pl.BlockSpec((RB, 128), row),
          pl.BlockSpec((RB, 128), row),
          pl.BlockSpec((RB, 128), row),
          pl.BlockSpec((RB, 128), row),
          pl.BlockSpec((RB, 1), row),
          pl.BlockSpec((1, 320), const),
          pl.BlockSpec((RB, 1), row),
          pl.BlockSpec((NB, 128), const),
          pl.BlockSpec((320, 1024), const),
          pl.BlockSpec((1, 1024), const),
          pl.BlockSpec((1024, 128), const),
          pl.BlockSpec((1, 128), const),
          pl.BlockSpec((128, 1024), const),
          pl.BlockSpec((128, 1024), const),
          pl.BlockSpec((1, 1024), const),
          pl.BlockSpec((1024, 512), const),
          pl.BlockSpec((1, 512), const),
          pl.BlockSpec((512, 1), const),
          pl.BlockSpec((1, 1), const),
      ],
      out_specs=pl.BlockSpec((NB, 1), const),
      out_shape=_SDS((NB, 1), _f32),
      scratch_shapes=[pltpu.VMEM((NB, 320), _f32), pltpu.VMEM((NB, 128), _f32)],
      compiler_params=pltpu.CompilerParams(dimension_semantics=("arbitrary",)),
  )(oa, ob, q0, q1, g2c, dinv, b3, batch1, fp_pad,
    wg1, bg1, wg2, bg2, wf1a, wf1b, bf1, wf2, bf2, wo, bo)


def kernel(x, edge_index, batch, data_fp, W1, b1, W2, b2, W3, b3,
           Wg1, bg1, Wg2, bg2, Wf1, bf1, Wf2, bf2, Wo, bo):
  pad_idx = (N + (jnp.arange(EP - E, dtype=jnp.int32) % (NP - N))).astype(jnp.int32)
  src2 = jnp.concatenate([edge_index[0], pad_idx]).reshape(ER, W)
  dst2 = jnp.concatenate([edge_index[1], pad_idx]).reshape(ER, W)
  x_pad = jnp.pad(x, ((0, NP - N), (0, 0)))
  batch1 = jnp.pad(batch, (0, NP - N), constant_values=NB).reshape(NP, 1)
  zeros_np = jnp.zeros((NP,), _f32)
  fp_pad = jnp.pad(data_fp.astype(_f32), ((0, 0), (0, 63)))
  wf1a = jnp.pad(Wf1[:65], ((0, 63), (0, 0)))
  wf1b = Wf1[65:]
  b1r = b1.reshape(1, 128)
  b2r = b2.reshape(1, 640)
  b3r = b3.reshape(1, 320)
  bg1r, bg2r = bg1.reshape(1, 1024), bg2.reshape(1, 128)
  bf1r, bf2r = bf1.reshape(1, 1024), bf2.reshape(1, 512)
  bor = bo.reshape(1, 1)

  dega, degb = _deg_kernel()(dst2, zeros_np)
  dinv, g0 = _k1(dega.reshape(NP, 1), degb.reshape(NP, 1), x_pad)

  p0, p1 = _agg_split_kernel()(g0, src2, dst2)
  g1 = _k2(p0, p1, g0, dinv, W1, b1r)

  p0, p1 = _agg_split_kernel()(g1, src2, dst2)
  g2a, g2b, g2c = _k3(p0, p1, g1, dinv, W2, b2r, W3)

  oa, ob = _agg_dual_kernel()(g2a, g2b, src2, dst2)
  q0, q1 = _agg_split_kernel()(g2c, src2, dst2)

  return _k45(oa, ob, q0, q1, g2c, dinv, b3r, batch1, fp_pad,
              Wg1, bg1r, Wg2, bg2r, wf1a, wf1b, bf1r, Wf2, bf2r, Wo, bor)


# speedup vs baseline: 28.5120x; 1.0130x over previous
import functools

import jax
import jax.numpy as jnp
from jax import lax
from jax.experimental import pallas as pl
from jax.experimental.pallas import tpu as pltpu
from jax.experimental.pallas import tpu_sc as plsc

N = 10000
NP = 10240
E = 320000
EP = 327680
W = 128
IDXB = 40
NBUF = 2
LA = 1
ER = EP // W
RPT = ER // 16
RPT_H = ER // 32
NB = 32
RB = 1024
NBLK = NP // RB

_f32 = jnp.float32
_SDS = jax.ShapeDtypeStruct


def _sc_mesh():
  return plsc.VectorSubcoreMesh(core_axis_name="c", subcore_axis_name="s")


def _deg_body(dst3, zeros_np, out_a, out_b, sdeg, dstbuf, ones_v, dsem):
  c = lax.axis_index("c")
  s = lax.axis_index("s")
  for i in range(W // 16):
    ones_v[pl.ds(i * 16, 16)] = jnp.full((16,), 1.0, _f32)
  pltpu.sync_copy(zeros_np.at[pl.ds(s * 640, 640)], sdeg.at[pl.ds(s * 640, 640)])
  pltpu.sync_copy(dst3.at[pl.ds(c * (RPT_H * 16) + s * RPT_H, RPT_H)], dstbuf)
  plsc.subcore_barrier()

  def step(j, carry):
    pltpu.async_copy(ones_v, sdeg.at[dstbuf.at[j]], dsem, add=True)
    return carry

  lax.fori_loop(0, RPT_H, step, 0)

  def drain(j, carry):
    pltpu.make_async_copy(ones_v, sdeg.at[dstbuf.at[0]], dsem).wait()
    return carry

  lax.fori_loop(0, RPT_H, drain, 0)
  plsc.subcore_barrier()

  @pl.when(c == 0)
  def _():
    pltpu.sync_copy(sdeg.at[pl.ds(s * 640, 640)], out_a.at[pl.ds(s * 640, 640)])

  @pl.when(c == 1)
  def _():
    pltpu.sync_copy(sdeg.at[pl.ds(s * 640, 640)], out_b.at[pl.ds(s * 640, 640)])


@functools.cache
def _deg_kernel():
  return pl.kernel(
      _deg_body,
      out_type=(_SDS((NP,), _f32), _SDS((NP,), _f32)),
      mesh=_sc_mesh(),
      scratch_types=[
          pltpu.VMEM_SHARED((NP,), _f32),
          pltpu.VMEM((RPT_H, W), jnp.int32),
          pltpu.VMEM((W,), _f32),
          pltpu.SemaphoreType.DMA,
      ],
  )


def _edge_pass(g_ref, acc, src3, dst3, base, nrows,
               srcblk, dstblk, bufs, gsems, ssems):
  def blk(bi, carry):
    off = base + bi * IDXB
    pltpu.sync_copy(src3.at[pl.ds(off, IDXB)], srcblk)
    pltpu.sync_copy(dst3.at[pl.ds(off, IDXB)], dstblk)
    for b in range(LA):
      @pl.when(bi > 0)
      def _():
        pltpu.make_async_copy(bufs[b], acc.at[dstblk.at[0]], ssems[b]).wait()
      pltpu.async_copy(g_ref.at[srcblk.at[b]], bufs[b], gsems[b])

    def grp(gi, c2):
      for b in range(NBUF):
        j = NBUF * gi + b
        jg = bi * IDXB + j
        b2 = (b + LA) % NBUF

        @pl.when(j + LA < IDXB)
        def _():
          @pl.when(jg >= NBUF - LA)
          def _():
            pltpu.make_async_copy(bufs[b2], acc.at[dstblk.at[j]],
                                  ssems[b2]).wait()
          pltpu.async_copy(g_ref.at[srcblk.at[j + LA]], bufs[b2], gsems[b2])

        pltpu.make_async_copy(g_ref.at[srcblk.at[j]], bufs[b], gsems[b]).wait()
        pltpu.async_copy(bufs[b], acc.at[dstblk.at[j]], ssems[b], add=True)
      return c2

    lax.fori_loop(0, IDXB // NBUF, grp, 0)
    return carry

  lax.fori_loop(0, nrows // IDXB, blk, 0)
  for b in range(NBUF):
    pltpu.make_async_copy(bufs[b], acc.at[dstblk.at[0]], ssems[b]).wait()


def _agg_split_body(g, src3, dst3, out0, out1,
                    acc, srcblk, dstblk, *bufs_sems):
  bufs = list(bufs_sems[:NBUF])
  gsems = list(bufs_sems[NBUF:2 * NBUF])
  ssems = list(bufs_sems[2 * NBUF:])
  c = lax.axis_index("c")
  s = lax.axis_index("s")
  pltpu.sync_copy(g.at[pl.ds(s * 640, 640)], acc.at[pl.ds(s * 640, 640)])
  plsc.subcore_barrier()
  _edge_pass(g, acc, src3, dst3, c * (RPT_H * 16) + s * RPT_H, RPT_H,
             srcblk, dstblk, bufs, gsems, ssems)
  plsc.subcore_barrier()

  @pl.when(c == 0)
  def _():
    pltpu.sync_copy(acc.at[pl.ds(s * 640, 640)], out0.at[pl.ds(s * 640, 640)])

  @pl.when(c == 1)
  def _():
    pltpu.sync_copy(acc.at[pl.ds(s * 640, 640)], out1.at[pl.ds(s * 640, 640)])


def _agg_l3_body(g_a, g_b, g_c, src3, dst3, out_a, out_b, q0, q1,
                 acc, srcblk, dstblk, *bufs_sems):
  bufs = list(bufs_sems[:NBUF])
  gsems = list(bufs_sems[NBUF:2 * NBUF])
  ssems = list(bufs_sems[2 * NBUF:])
  c = lax.axis_index("c")
  s = lax.axis_index("s")

  def run(g_ref, out_ref, base, nrows):
    pltpu.sync_copy(g_ref.at[pl.ds(s * 640, 640)], acc.at[pl.ds(s * 640, 640)])
    plsc.subcore_barrier()
    _edge_pass(g_ref, acc, src3, dst3, base, nrows,
               srcblk, dstblk, bufs, gsems, ssems)
    plsc.subcore_barrier()
    pltpu.sync_copy(acc.at[pl.ds(s * 640, 640)], out_ref.at[pl.ds(s * 640, 640)])

  @pl.when(c == 0)
  def _():
    run(g_a, out_a, s * RPT, RPT)
    run(g_c, q0, s * RPT_H, RPT_H)

  @pl.when(c == 1)
  def _():
    run(g_b, out_b, s * RPT, RPT)
    run(g_c, q1, RPT_H * 16 + s * RPT_H, RPT_H)


def _agg_scratch():
  return ([pltpu.VMEM_SHARED((NP, 128), _f32),
           pltpu.VMEM((IDXB, W), jnp.int32),
           pltpu.VMEM((IDXB, W), jnp.int32)]
          + [pltpu.VMEM((W, 128), _f32)] * NBUF
          + [pltpu.SemaphoreType.DMA] * (2 * NBUF))


@functools.cache
def _agg_split_kernel():
  return pl.kernel(
      _agg_split_body,
      out_type=(_SDS((NP, 128), _f32), _SDS((NP, 128), _f32)),
      mesh=_sc_mesh(),
      scratch_types=_agg_scratch(),
  )


@functools.cache
def _agg_l3_kernel():
  return pl.kernel(
      _agg_l3_body,
      out_type=tuple([_SDS((NP, 128), _f32)] * 4),
      mesh=_sc_mesh(),
      scratch_types=_agg_scratch(),
  )


def _row_mask(i):
  rid = i * RB + lax.broadcasted_iota(jnp.int32, (RB, 1), 0)
  return (rid < N).astype(_f32)


def _k1_body(dega_ref, degb_ref, x_ref, dinv_ref, g_ref):
  i = pl.program_id(0)
  deg = dega_ref[...] + degb_ref[...] + 1.0
  dinv = lax.rsqrt(jnp.maximum(deg, 1.0))
  dinv_ref[...] = dinv
  g_ref[...] = x_ref[...] * dinv * _row_mask(i)


def _k1(dega, degb, x2d):
  return pl.pallas_call(
      _k1_body,
      grid=(NBLK,),
      in_specs=[
          pl.BlockSpec((RB, 1), lambda i: (i, 0)),
          pl.BlockSpec((RB, 1), lambda i: (i, 0)),
          pl.BlockSpec((RB, 128), lambda i: (i, 0)),
      ],
      out_specs=[
          pl.BlockSpec((RB, 1), lambda i: (i, 0)),
          pl.BlockSpec((RB, 128), lambda i: (i, 0)),
      ],
      out_shape=[_SDS((NP, 1), _f32), _SDS((NP, 128), _f32)],
  )(dega, degb, x2d)


def _k2_body(p0_ref, p1_ref, g_ref, dinv_ref, w_ref, b_ref, out_ref):
  i = pl.program_id(0)
  d = dinv_ref[...]
  sm = (p0_ref[...] + p1_ref[...] - g_ref[...]) * d
  t = jnp.dot(sm, w_ref[...], preferred_element_type=_f32) + b_ref[...]
  out_ref[...] = jnp.maximum(t, 0.0) * d * _row_mask(i)


def _k2(p0, p1, g0, dinv, w1, b1):
  return pl.pallas_call(
      _k2_body,
      grid=(NBLK,),
      in_specs=[
          pl.BlockSpec((RB, 128), lambda i: (i, 0)),
          pl.BlockSpec((RB, 128), lambda i: (i, 0)),
          pl.BlockSpec((RB, 128), lambda i: (i, 0)),
          pl.BlockSpec((RB, 1), lambda i: (i, 0)),
          pl.BlockSpec((128, 128), lambda i: (0, 0)),
          pl.BlockSpec((1, 128), lambda i: (0, 0)),
      ],
      out_specs=pl.BlockSpec((RB, 128), lambda i: (i, 0)),
      out_shape=_SDS((NP, 128), _f32),
  )(p0, p1, g0, dinv, w1, b1)


def _k3_body(p0_ref, p1_ref, g_ref, dinv_ref, w2_ref, b2_ref, w3_ref,
             ga_ref, gb_ref, gc_ref):
  i = pl.program_id(0)
  d = dinv_ref[...]
  sm = (p0_ref[...] + p1_ref[...] - g_ref[...]) * d
  u = jnp.dot(sm, w2_ref[...], preferred_element_type=_f32) + b2_ref[...]
  u = jnp.maximum(u, 0.0)
  h = jnp.dot(u, w3_ref[...], preferred_element_type=_f32)
  g = h * d * _row_mask(i)
  ga_ref[...] = g[:, :128]
  gb_ref[...] = g[:, 128:256]
  gc_ref[...] = g[:, 192:]


def _k3(p0, p1, g1, dinv, w2, b2, w3):
  return pl.pallas_call(
      _k3_body,
      grid=(NBLK,),
      in_specs=[
          pl.BlockSpec((RB, 128), lambda i: (i, 0)),
          pl.BlockSpec((RB, 128), lambda i: (i, 0)),
          pl.BlockSpec((RB, 128), lambda i: (i, 0)),
          pl.BlockSpec((RB, 1), lambda i: (i, 0)),
          pl.BlockSpec((128, 640), lambda i: (0, 0)),
          pl.BlockSpec((1, 640), lambda i: (0, 0)),
          pl.BlockSpec((640, 320), lambda i: (0, 0)),
      ],
      out_specs=[
          pl.BlockSpec((RB, 128), lambda i: (i, 0)),
          pl.BlockSpec((RB, 128), lambda i: (i, 0)),
          pl.BlockSpec((RB, 128), lambda i: (i, 0)),
      ],
      out_shape=[_SDS((NP, 128), _f32)] * 3,
  )(p0, p1, g1, dinv, w2, b2, w3)


def _k45_body(oa_ref, ob_ref, q0_ref, q1_ref, gc_ref, dinv_ref, b3_ref,
              batch_ref, fp_ref, wg1_ref, bg1_ref, wg2_ref, bg2_ref,
              wf1a_ref, wf1b_ref, bf1_ref, wf2_ref, bf2_ref, wo_ref, bo_ref,
              out_ref, sums_ref, cnt_ref):
  i = pl.program_id(0)

  @pl.when(i < NBLK)
  def _():
    d = dinv_ref[...]
    sc2 = q0_ref[...] + q1_ref[...] - gc_ref[...]
    s2 = jnp.concatenate([oa_ref[...], ob_ref[:, :64], sc2], axis=1)
    x3 = jnp.maximum(s2 * d + b3_ref[...], 0.0)
    seg = lax.broadcasted_iota(jnp.int32, (RB, NB), 1)
    oh = (batch_ref[...] == seg).astype(_f32)
    psum = lax.dot_general(oh, x3, (((0,), (0,)), ((), ())),
                           preferred_element_type=_f32)
    pcnt = lax.dot_general(oh, jnp.ones((RB, 128), _f32),
                           (((0,), (0,)), ((), ())),
                           preferred_element_type=_f32)

    @pl.when(i == 0)
    def _():
      sums_ref[...] = psum
      cnt_ref[...] = pcnt

    @pl.when(i > 0)
    def _():
      sums_ref[...] += psum
      cnt_ref[...] += pcnt

  @pl.when(i == NBLK)
  def _():
    cnt = jnp.maximum(cnt_ref[:, :1], 1.0)
    mean = sums_ref[...] / cnt
    z1 = jnp.maximum(jnp.dot(mean, wg1_ref[...], preferred_element_type=_f32)
                     + bg1_ref[...], 0.0)
    z2 = jnp.dot(z1, wg2_ref[...], preferred_element_type=_f32) + bg2_ref[...]
    t = (jnp.dot(fp_ref[...], wf1a_ref[...], preferred_element_type=_f32)
         + jnp.dot(z2, wf1b_ref[...], preferred_element_type=_f32)
         + bf1_ref[...])
    t = jnp.maximum(t, 0.0)
    t2 = jnp.maximum(jnp.dot(t, wf2_ref[...], preferred_element_type=_f32)
                     + bf2_ref[...], 0.0)
    out_ref[...] = (jnp.dot(t2, wo_ref[...], preferred_element_type=_f32)
                    + bo_ref[...])


def _k45(oa, ob, q0, q1, g2c, dinv, b3, batch1, fp_pad,
         wg1, bg1, wg2, bg2, wf1a, wf1b, bf1, wf2, bf2, wo, bo):
  row = lambda i: (jnp.minimum(i, NBLK - 1), 0)
  const = lambda i: (0, 0)
  return pl.pallas_call(
      _k45_body,
      grid=(NBLK + 1,),
      in_specs=[
          pl.BlockSpec((RB, 128), row),
          pl.BlockSpec((RB, 128), row),
          pl.BlockSpec((RB, 128), row),
          pl.BlockSpec((RB, 128), row),
          pl.BlockSpec((RB, 128), row),
          pl.BlockSpec((RB, 1), row),
          pl.BlockSpec((1, 320), const),
          pl.BlockSpec((RB, 1), row),
          pl.BlockSpec((NB, 128), const),
          pl.BlockSpec((320, 1024), const),
          pl.BlockSpec((1, 1024), const),
          pl.BlockSpec((1024, 128), const),
          pl.BlockSpec((1, 128), const),
          pl.BlockSpec((128, 1024), const),
          pl.BlockSpec((128, 1024), const),
          pl.BlockSpec((1, 1024), const),
          pl.BlockSpec((1024, 512), const),
          pl.BlockSpec((1, 512), const),
          pl.BlockSpec((512, 1), const),
          pl.BlockSpec((1, 1), const),
      ],
      out_specs=pl.BlockSpec((NB, 1), const),
      out_shape=_SDS((NB, 1), _f32),
      scratch_shapes=[pltpu.VMEM((NB, 320), _f32), pltpu.VMEM((NB, 128), _f32)],
      compiler_params=pltpu.CompilerParams(dimension_semantics=("arbitrary",)),
  )(oa, ob, q0, q1, g2c, dinv, b3, batch1, fp_pad,
    wg1, bg1, wg2, bg2, wf1a, wf1b, bf1, wf2, bf2, wo, bo)


def kernel(x, edge_index, batch, data_fp, W1, b1, W2, b2, W3, b3,
           Wg1, bg1, Wg2, bg2, Wf1, bf1, Wf2, bf2, Wo, bo):
  pad_idx = (N + (jnp.arange(EP - E, dtype=jnp.int32) % (NP - N))).astype(jnp.int32)
  src2 = jnp.concatenate([edge_index[0], pad_idx]).reshape(ER, W)
  dst2 = jnp.concatenate([edge_index[1], pad_idx]).reshape(ER, W)
  x2d = jnp.pad(x, ((0, NP - N), (0, 0)))
  batch1 = jnp.pad(batch, (0, NP - N), constant_values=NB).reshape(NP, 1)
  zeros_np = jnp.zeros((NP,), _f32)
  fp_pad = jnp.pad(data_fp.astype(_f32), ((0, 0), (0, 63)))
  wf1a = jnp.pad(Wf1[:65], ((0, 63), (0, 0)))
  wf1b = Wf1[65:]
  b1r = b1.reshape(1, 128)
  b2r = b2.reshape(1, 640)
  b3r = b3.reshape(1, 320)
  bg1r, bg2r = bg1.reshape(1, 1024), bg2.reshape(1, 128)
  bf1r, bf2r = bf1.reshape(1, 1024), bf2.reshape(1, 512)
  bor = bo.reshape(1, 1)

  dega, degb = _deg_kernel()(dst2, zeros_np)
  dinv, g0 = _k1(dega.reshape(NP, 1), degb.reshape(NP, 1), x2d)

  p0, p1 = _agg_split_kernel()(g0, src2, dst2)
  g1 = _k2(p0, p1, g0, dinv, W1, b1r)

  p0, p1 = _agg_split_kernel()(g1, src2, dst2)
  g2a, g2b, g2c = _k3(p0, p1, g1, dinv, W2, b2r, W3)

  oa, ob, q0, q1 = _agg_l3_kernel()(g2a, g2b, g2c, src2, dst2)

  return _k45(oa, ob, q0, q1, g2c, dinv, b3r, batch1, fp_pad,
              Wg1, bg1r, Wg2, bg2r, wf1a, wf1b, bf1r, Wf2, bf2r, Wo, bor)
